# Initial kernel scaffold; baseline (speedup 1.0000x reference)
import functools

import jax
import jax.numpy as jnp
from jax import lax
from jax.experimental import pallas as pl
from jax.experimental.pallas import tpu as pltpu
from jax.experimental.pallas import tpu_sc as plsc

F32 = jnp.float32

_HID = 128
_HEADS = 8
_HDIM = 16

_NC = 2
_NS = 16
_NW = _NC * _NS

_KB = 80
_ZB = 128


def _leaky(z):
    return jnp.where(z >= 0, z, 0.2 * z)


def _lnorm(v, g, b):
    mu = jnp.mean(v, axis=-1, keepdims=True)
    var = jnp.mean((v - mu) * (v - mu), axis=-1, keepdims=True)
    return (v - mu) * lax.rsqrt(var + 1e-5) * g + b


def _gelu(v):
    return 0.5 * v * (1.0 + lax.erf(v * 0.7071067811865476))


def _proj(x, WlT, bl2, WrT, br2):
    n = x.shape[0]
    bn = 1000
    grid = (n // bn,)

    def body(x_r, wl_r, bl_r, wr_r, br_r, xl_r, xr_r):
        xv = x_r[...]
        xl_r[...] = jnp.dot(xv, wl_r[...], preferred_element_type=F32) + bl_r[...]
        xr_r[...] = jnp.dot(xv, wr_r[...], preferred_element_type=F32) + br_r[...]

    full = pl.BlockSpec((_HID, _HID), lambda i: (0, 0))
    vec = pl.BlockSpec((1, _HID), lambda i: (0, 0))
    rows = pl.BlockSpec((bn, _HID), lambda i: (i, 0))
    return pl.pallas_call(
        body,
        grid=grid,
        in_specs=[rows, full, vec, full, vec],
        out_specs=[rows, rows],
        out_shape=[
            jax.ShapeDtypeStruct((n, _HID), F32),
            jax.ShapeDtypeStruct((n, _HID), F32),
        ],
    )(x, WlT, bl2, WrT, br2)


def _sc_gather(e, np_):
    epw = e // _NW
    nblk = epw // _KB
    rpt = np_ // _NS
    mesh = plsc.VectorSubcoreMesh(core_axis_name="c", subcore_axis_name="s")

    @functools.partial(
        pl.kernel,
        out_type=(
            jax.ShapeDtypeStruct((e, _HID), F32),
            jax.ShapeDtypeStruct((e, _HID), F32),
            jax.ShapeDtypeStruct((_NC, np_, _HID), F32),
        ),
        mesh=mesh,
        scratch_types=(
            pltpu.VMEM((_KB,), jnp.int32),
            pltpu.VMEM((_KB,), jnp.int32),
            pltpu.VMEM((_KB, _HID), F32),
            pltpu.VMEM((_KB, _HID), F32),
            pltpu.VMEM((_KB, _HID), F32),
            pltpu.VMEM((_ZB, _HID), F32),
            pltpu.VMEM_SHARED((np_, _HID), F32),
            pltpu.SemaphoreType.DMA,
            pltpu.SemaphoreType.DMA,
        ),
    )
    def k(xl_hbm, xr_hbm, src_hbm, dst_hbm, ea_hbm,
          xlg_out, xrg_out, ls_out,
          src_v, dst_v, xlr, xrr, ear, zbuf, ls_sh, sem1, sem2):
        c = lax.axis_index("c")
        s = lax.axis_index("s")
        wid = c * _NS + s

        def zrow(i, _):
            for h in range(_HID // 16):
                zbuf[i, pl.ds(h * 16, 16)] = jnp.zeros((16,), F32)
            return 0
        lax.fori_loop(0, _ZB, zrow, 0)

        def zchunk(j, _):
            pltpu.sync_copy(zbuf, ls_sh.at[pl.ds(s * rpt + j * _ZB, _ZB)])
            return 0
        lax.fori_loop(0, rpt // _ZB, zchunk, 0)
        plsc.subcore_barrier()

        base0 = wid * epw

        def body(i, _):
            base = base0 + i * _KB
            pltpu.sync_copy(src_hbm.at[pl.ds(base, _KB)], src_v)
            pltpu.sync_copy(dst_hbm.at[pl.ds(base, _KB)], dst_v)
            cp1 = pltpu.async_copy(xl_hbm.at[src_v], xlr, sem1)
            cp2 = pltpu.async_copy(xr_hbm.at[dst_v], xrr, sem2)
            pltpu.sync_copy(ea_hbm.at[pl.ds(base, _KB)], ear)
            cp1.wait()
            cp2.wait()
            pltpu.sync_copy(xlr, xlg_out.at[pl.ds(base, _KB)])
            pltpu.sync_copy(xrr, xrg_out.at[pl.ds(base, _KB)])
            pltpu.sync_copy(ear, ls_sh.at[dst_v], add=True)
            return 0
        lax.fori_loop(0, nblk, body, 0)
        plsc.subcore_barrier()

        def wchunk(j, _):
            off = s * rpt + j * _ZB
            pltpu.sync_copy(ls_sh.at[pl.ds(off, _ZB)],
                            ls_out.at[c, pl.ds(off, _ZB)])
            return 0
        lax.fori_loop(0, rpt // _ZB, wchunk, 0)

    return k


def _edge_math(ea, xlg, xrg, WeT, A, S):
    e = ea.shape[0]
    be = 2000
    grid = (e // be,)

    def body(ea_r, xlg_r, xrg_r, wet_r, a_r, s_r, ae_r, msg_r):
        xlv = xlg_r[...]
        ee = jnp.dot(ea_r[...], wet_r[...], preferred_element_type=F32)
        z = xlv + xrg_r[...] + ee
        l16 = jnp.dot(_leaky(z), a_r[...], preferred_element_type=F32)
        aev = jnp.exp(l16)
        ae_r[...] = aev
        msg_r[...] = xlv * jnp.dot(aev, s_r[...], preferred_element_type=F32)

    rows = pl.BlockSpec((be, _HID), lambda i: (i, 0))
    return pl.pallas_call(
        body,
        grid=grid,
        in_specs=[
            rows, rows, rows,
            pl.BlockSpec((_HID, _HID), lambda i: (0, 0)),
            pl.BlockSpec((_HID, 16), lambda i: (0, 0)),
            pl.BlockSpec((16, _HID), lambda i: (0, 0)),
        ],
        out_specs=[pl.BlockSpec((be, 16), lambda i: (i, 0)), rows],
        out_shape=[
            jax.ShapeDtypeStruct((e, 16), F32),
            jax.ShapeDtypeStruct((e, _HID), F32),
        ],
    )(ea, xlg, xrg, WeT, A, S)


def _sc_scatter(e, np_):
    epw = e // _NW
    nblk = epw // _KB
    rpt = np_ // _NS
    mesh = plsc.VectorSubcoreMesh(core_axis_name="c", subcore_axis_name="s")

    @functools.partial(
        pl.kernel,
        out_type=(
            jax.ShapeDtypeStruct((_NC, np_, _HID), F32),
            jax.ShapeDtypeStruct((_NC, np_, 16), F32),
        ),
        mesh=mesh,
        scratch_types=(
            pltpu.VMEM((_KB,), jnp.int32),
            pltpu.VMEM((_KB, _HID), F32),
            pltpu.VMEM((_KB, 16), F32),
            pltpu.VMEM((_ZB, _HID), F32),
            pltpu.VMEM((_ZB, 16), F32),
            pltpu.VMEM_SHARED((np_, _HID), F32),
            pltpu.VMEM_SHARED((np_, 16), F32),
        ),
    )
    def k(dst_hbm, msg_hbm, ae_hbm,
          num_out, den_out,
          dst_v, msgr, aer, zbuf, zbuf16, num_sh, den_sh):
        c = lax.axis_index("c")
        s = lax.axis_index("s")
        wid = c * _NS + s

        def zrow(i, _):
            for h in range(_HID // 16):
                zbuf[i, pl.ds(h * 16, 16)] = jnp.zeros((16,), F32)
            zbuf16[i, pl.ds(0, 16)] = jnp.zeros((16,), F32)
            return 0
        lax.fori_loop(0, _ZB, zrow, 0)

        def zchunk(j, _):
            off = s * rpt + j * _ZB
            pltpu.sync_copy(zbuf, num_sh.at[pl.ds(off, _ZB)])
            pltpu.sync_copy(zbuf16, den_sh.at[pl.ds(off, _ZB)])
            return 0
        lax.fori_loop(0, rpt // _ZB, zchunk, 0)
        plsc.subcore_barrier()

        base0 = wid * epw

        def body(i, _):
            base = base0 + i * _KB
            pltpu.sync_copy(dst_hbm.at[pl.ds(base, _KB)], dst_v)
            pltpu.sync_copy(msg_hbm.at[pl.ds(base, _KB)], msgr)
            pltpu.sync_copy(ae_hbm.at[pl.ds(base, _KB)], aer)
            pltpu.sync_copy(msgr, num_sh.at[dst_v], add=True)
            pltpu.sync_copy(aer, den_sh.at[dst_v], add=True)
            return 0
        lax.fori_loop(0, nblk, body, 0)
        plsc.subcore_barrier()

        def wchunk(j, _):
            off = s * rpt + j * _ZB
            pltpu.sync_copy(num_sh.at[pl.ds(off, _ZB)],
                            num_out.at[c, pl.ds(off, _ZB)])
            pltpu.sync_copy(den_sh.at[pl.ds(off, _ZB)],
                            den_out.at[c, pl.ds(off, _ZB)])
            return 0
        lax.fori_loop(0, rpt // _ZB, wchunk, 0)

    return k


def _finish(x, xl, xr, ls0, ls1, num0, num1, den0, den1,
            WeT, A, S, W1T, b1r, W2T, b2r, bor, g1r, be1r, g2r, be2r):
    n = x.shape[0]
    bn = 1000
    grid = (n // bn,)

    def body(x_r, xl_r, xr_r, ls0_r, ls1_r, num0_r, num1_r, den0_r, den1_r,
             wet_r, a_r, s_r, w1_r, b1_r, w2_r, b2_r, bo_r,
             g1_r, be1_r, g2_r, be2_r, out_r):
        xlv = xl_r[...]
        den16 = den0_r[...] + den1_r[...]
        deg = jnp.maximum(den16[:, 8:9], 1.0)
        la = (ls0_r[...] + ls1_r[...]) / deg
        lee = jnp.dot(la, wet_r[...], preferred_element_type=F32)
        z = xlv + xr_r[...] + lee
        a16 = jnp.exp(jnp.dot(_leaky(z), a_r[...], preferred_element_type=F32))
        den_e = jnp.dot(den16 + a16, s_r[...], preferred_element_type=F32)
        num_e = (num0_r[...] + num1_r[...]
                 + xlv * jnp.dot(a16, s_r[...], preferred_element_type=F32))
        attn = num_e / den_e + bo_r[...]
        h1 = _lnorm(x_r[...] + attn, g1_r[...], be1_r[...])
        p = jnp.dot(h1, w1_r[...], preferred_element_type=F32) + b1_r[...]
        f = jnp.dot(_gelu(p), w2_r[...], preferred_element_type=F32) + b2_r[...]
        out_r[...] = _lnorm(h1 + f, g2_r[...], be2_r[...])

    rows = pl.BlockSpec((bn, _HID), lambda i: (i, 0))
    rows16 = pl.BlockSpec((bn, 16), lambda i: (i, 0))
    full = pl.BlockSpec((_HID, _HID), lambda i: (0, 0))
    vec = pl.BlockSpec((1, _HID), lambda i: (0, 0))
    return pl.pallas_call(
        body,
        grid=grid,
        in_specs=[
            rows, rows, rows, rows, rows, rows, rows, rows16, rows16,
            full,
            pl.BlockSpec((_HID, 16), lambda i: (0, 0)),
            pl.BlockSpec((16, _HID), lambda i: (0, 0)),
            pl.BlockSpec((_HID, 4 * _HID), lambda i: (0, 0)),
            pl.BlockSpec((1, 4 * _HID), lambda i: (0, 0)),
            pl.BlockSpec((4 * _HID, _HID), lambda i: (0, 0)),
            vec, vec, vec, vec, vec, vec,
        ],
        out_specs=rows,
        out_shape=jax.ShapeDtypeStruct((n, _HID), F32),
    )(x, xl, xr, ls0, ls1, num0, num1, den0, den1,
      WeT, A, S, W1T, b1r, W2T, b2r, bor, g1r, be1r, g2r, be2r)


def kernel(x, edge_index, edge_attr, batch, Wl, bl, Wr, br, We, att,
           bias_out, W1, b1, W2, b2, g1, be1, g2, be2):
    n = x.shape[0]
    e = edge_index.shape[1]
    src = edge_index[0]
    dst = edge_index[1]

    np_ = -(-n // (_NS * _ZB)) * (_NS * _ZB)

    a8 = att.reshape(_HEADS, _HDIM)
    eye8 = jnp.eye(_HEADS, dtype=F32)
    amat = (a8[:, :, None] * eye8[:, None, :]).reshape(_HEADS * _HDIM, _HEADS)
    amat = jnp.pad(amat, ((0, 0), (0, 16 - _HEADS)))
    smat = jnp.pad(jnp.repeat(jnp.eye(_HEADS, dtype=F32), _HDIM, axis=1),
                   ((0, 16 - _HEADS), (0, 0)))

    xl, xr = _proj(x, Wl.T, bl.reshape(1, -1), Wr.T, br.reshape(1, -1))
    xlg, xrg, lsp = _sc_gather(e, np_)(xl, xr, src, dst, edge_attr)
    ae, msg = _edge_math(edge_attr, xlg, xrg, We.T, amat, smat)
    nump, denp = _sc_scatter(e, np_)(dst, msg, ae)
    return _finish(
        x, xl, xr, lsp[0, :n], lsp[1, :n], nump[0, :n], nump[1, :n],
        denp[0, :n], denp[1, :n],
        We.T, amat, smat, W1.T, b1.reshape(1, -1), W2.T, b2.reshape(1, -1),
        bias_out.reshape(1, -1), g1.reshape(1, -1), be1.reshape(1, -1),
        g2.reshape(1, -1), be2.reshape(1, -1))


# speedup vs baseline: 32.4958x; 32.4958x over previous
import functools

import jax
import jax.numpy as jnp
from jax import lax
from jax.experimental import pallas as pl
from jax.experimental.pallas import tpu as pltpu
from jax.experimental.pallas import tpu_sc as plsc

F32 = jnp.float32

_HID = 128
_HEADS = 8
_HDIM = 16

_NC = 2
_NS = 16
_NW = _NC * _NS

_KB = 80


def _leaky(z):
    return jnp.where(z >= 0, z, 0.2 * z)


def _lnorm(v, g, b):
    mu = jnp.mean(v, axis=-1, keepdims=True)
    var = jnp.mean((v - mu) * (v - mu), axis=-1, keepdims=True)
    return (v - mu) * lax.rsqrt(var + 1e-5) * g + b


def _gelu(v):
    return 0.5 * v * (1.0 + lax.erf(v * 0.7071067811865476))


def _proj(x, WlT, bl2, WrT, br2):
    n = x.shape[0]
    bn = 1000
    grid = (n // bn,)

    def body(x_r, wl_r, bl_r, wr_r, br_r, xl_r, xr_r):
        xv = x_r[...]
        xl_r[...] = jnp.dot(xv, wl_r[...], preferred_element_type=F32) + bl_r[...]
        xr_r[...] = jnp.dot(xv, wr_r[...], preferred_element_type=F32) + br_r[...]

    full = pl.BlockSpec((_HID, _HID), lambda i: (0, 0))
    vec = pl.BlockSpec((1, _HID), lambda i: (0, 0))
    rows = pl.BlockSpec((bn, _HID), lambda i: (i, 0))
    return pl.pallas_call(
        body,
        grid=grid,
        in_specs=[rows, full, vec, full, vec],
        out_specs=[rows, rows],
        out_shape=[
            jax.ShapeDtypeStruct((n, _HID), F32),
            jax.ShapeDtypeStruct((n, _HID), F32),
        ],
    )(x, WlT, bl2, WrT, br2)


def _sc_gather(e, np_):
    epw = e // _NW
    nblk = epw // _KB
    rpt = np_ // _NS
    mesh = plsc.VectorSubcoreMesh(core_axis_name="c", subcore_axis_name="s")

    @functools.partial(
        pl.kernel,
        out_type=(
            jax.ShapeDtypeStruct((e, _HID), F32),
            jax.ShapeDtypeStruct((e, _HID), F32),
            jax.ShapeDtypeStruct((_NC, np_, _HID), F32),
        ),
        mesh=mesh,
        scratch_types=(
            pltpu.VMEM((_KB,), jnp.int32),
            pltpu.VMEM((_KB,), jnp.int32),
            pltpu.VMEM((_KB, _HID), F32),
            pltpu.VMEM((_KB, _HID), F32),
            pltpu.VMEM((_KB, _HID), F32),
            pltpu.VMEM_SHARED((np_, _HID), F32),
            pltpu.SemaphoreType.DMA,
            pltpu.SemaphoreType.DMA,
        ),
    )
    def k(xl_hbm, xr_hbm, src_hbm, dst_hbm, ea_hbm,
          xlg_out, xrg_out, ls_out,
          src_v, dst_v, xlr, xrr, ear, ls_sh, sem1, sem2):
        c = lax.axis_index("c")
        s = lax.axis_index("s")
        wid = c * _NS + s

        def zrow(i, _):
            for h in range(_HID // 16):
                ear[i, pl.ds(h * 16, 16)] = jnp.zeros((16,), F32)
            return 0
        lax.fori_loop(0, _KB, zrow, 0)

        def zchunk(j, _):
            pltpu.sync_copy(ear, ls_sh.at[pl.ds(s * rpt + j * _KB, _KB)])
            return 0
        lax.fori_loop(0, rpt // _KB, zchunk, 0)
        plsc.subcore_barrier()

        base0 = wid * epw

        def body(i, _):
            base = base0 + i * _KB
            pltpu.sync_copy(src_hbm.at[pl.ds(base, _KB)], src_v)
            pltpu.sync_copy(dst_hbm.at[pl.ds(base, _KB)], dst_v)
            cp1 = pltpu.async_copy(xl_hbm.at[src_v], xlr, sem1)
            cp2 = pltpu.async_copy(xr_hbm.at[dst_v], xrr, sem2)
            pltpu.sync_copy(ea_hbm.at[pl.ds(base, _KB)], ear)
            cp1.wait()
            cp2.wait()
            pltpu.sync_copy(xlr, xlg_out.at[pl.ds(base, _KB)])
            pltpu.sync_copy(xrr, xrg_out.at[pl.ds(base, _KB)])
            pltpu.sync_copy(ear, ls_sh.at[dst_v], add=True)
            return 0
        lax.fori_loop(0, nblk, body, 0)
        plsc.subcore_barrier()

        def wchunk(j, _):
            off = s * rpt + j * _KB
            pltpu.sync_copy(ls_sh.at[pl.ds(off, _KB)],
                            ls_out.at[c, pl.ds(off, _KB)])
            return 0
        lax.fori_loop(0, rpt // _KB, wchunk, 0)

    return k


def _edge_math(ea, xlg, xrg, WeT, A, S):
    e = ea.shape[0]
    be = 2000
    grid = (e // be,)

    def body(ea_r, xlg_r, xrg_r, wet_r, a_r, s_r, ae_r, msg_r):
        xlv = xlg_r[...]
        ee = jnp.dot(ea_r[...], wet_r[...], preferred_element_type=F32)
        z = xlv + xrg_r[...] + ee
        l16 = jnp.dot(_leaky(z), a_r[...], preferred_element_type=F32)
        aev = jnp.exp(l16)
        ae_r[...] = aev
        msg_r[...] = xlv * jnp.dot(aev, s_r[...], preferred_element_type=F32)

    rows = pl.BlockSpec((be, _HID), lambda i: (i, 0))
    return pl.pallas_call(
        body,
        grid=grid,
        in_specs=[
            rows, rows, rows,
            pl.BlockSpec((_HID, _HID), lambda i: (0, 0)),
            pl.BlockSpec((_HID, 16), lambda i: (0, 0)),
            pl.BlockSpec((16, _HID), lambda i: (0, 0)),
        ],
        out_specs=[pl.BlockSpec((be, 16), lambda i: (i, 0)), rows],
        out_shape=[
            jax.ShapeDtypeStruct((e, 16), F32),
            jax.ShapeDtypeStruct((e, _HID), F32),
        ],
    )(ea, xlg, xrg, WeT, A, S)


def _sc_scatter(e, np_):
    epw = e // _NW
    nblk = epw // _KB
    rpt = np_ // _NS
    mesh = plsc.VectorSubcoreMesh(core_axis_name="c", subcore_axis_name="s")

    @functools.partial(
        pl.kernel,
        out_type=jax.ShapeDtypeStruct((_NC, np_, _HID), F32),
        mesh=mesh,
        scratch_types=(
            pltpu.VMEM((_KB,), jnp.int32),
            pltpu.VMEM((_KB, _HID), F32),
            pltpu.VMEM_SHARED((np_, _HID), F32),
        ),
    )
    def k(dst_hbm, msg_hbm, num_out, dst_v, msgr, num_sh):
        c = lax.axis_index("c")
        s = lax.axis_index("s")
        wid = c * _NS + s

        def zrow(i, _):
            for h in range(_HID // 16):
                msgr[i, pl.ds(h * 16, 16)] = jnp.zeros((16,), F32)
            return 0
        lax.fori_loop(0, _KB, zrow, 0)

        def zchunk(j, _):
            off = s * rpt + j * _KB
            pltpu.sync_copy(msgr, num_sh.at[pl.ds(off, _KB)])
            return 0
        lax.fori_loop(0, rpt // _KB, zchunk, 0)
        plsc.subcore_barrier()

        base0 = wid * epw

        def body(i, _):
            base = base0 + i * _KB
            pltpu.sync_copy(dst_hbm.at[pl.ds(base, _KB)], dst_v)
            pltpu.sync_copy(msg_hbm.at[pl.ds(base, _KB)], msgr)
            pltpu.sync_copy(msgr, num_sh.at[dst_v], add=True)
            return 0
        lax.fori_loop(0, nblk, body, 0)
        plsc.subcore_barrier()

        def wchunk(j, _):
            off = s * rpt + j * _KB
            pltpu.sync_copy(num_sh.at[pl.ds(off, _KB)],
                            num_out.at[c, pl.ds(off, _KB)])
            return 0
        lax.fori_loop(0, rpt // _KB, wchunk, 0)

    return k


def _sc_scatter_den(e, np_):
    epw = e // _NW
    nblk = epw // _KB
    rpt = np_ // _NS
    mesh = plsc.VectorSubcoreMesh(core_axis_name="c", subcore_axis_name="s")

    @functools.partial(
        pl.kernel,
        out_type=jax.ShapeDtypeStruct((_NC, np_, _HID), F32),
        mesh=mesh,
        scratch_types=(
            pltpu.VMEM((_KB,), jnp.int32),
            pltpu.VMEM((_KB * 16,), F32),
            pltpu.VMEM((_KB, _HID), F32),
            pltpu.VMEM_SHARED((np_, _HID), F32),
        ),
    )
    def k(dst_hbm, aef_hbm, den_out, dst_v, aebuf, aer, den_sh):
        c = lax.axis_index("c")
        s = lax.axis_index("s")
        wid = c * _NS + s

        def zrow(i, _):
            for h in range(_HID // 16):
                aer[i, pl.ds(h * 16, 16)] = jnp.zeros((16,), F32)
            return 0
        lax.fori_loop(0, _KB, zrow, 0)

        def zchunk(j, _):
            off = s * rpt + j * _KB
            pltpu.sync_copy(aer, den_sh.at[pl.ds(off, _KB)])
            return 0
        lax.fori_loop(0, rpt // _KB, zchunk, 0)
        plsc.subcore_barrier()

        base0 = wid * epw

        def body(i, _):
            base = base0 + i * _KB
            pltpu.sync_copy(dst_hbm.at[pl.ds(base, _KB)], dst_v)
            pltpu.sync_copy(aef_hbm.at[pl.ds(base * 16, _KB * 16)], aebuf)
            for r in range(_KB):
                aer[r, pl.ds(0, 16)] = aebuf[pl.ds(r * 16, 16)]
            pltpu.sync_copy(aer, den_sh.at[dst_v], add=True)
            return 0
        lax.fori_loop(0, nblk, body, 0)
        plsc.subcore_barrier()

        def wchunk(j, _):
            off = s * rpt + j * _KB
            pltpu.sync_copy(den_sh.at[pl.ds(off, _KB)],
                            den_out.at[c, pl.ds(off, _KB)])
            return 0
        lax.fori_loop(0, rpt // _KB, wchunk, 0)

    return k


def _finish(x, xl, xr, ls0, ls1, num0, num1, den0, den1,
            WeT, A, S, W1T, b1r, W2T, b2r, bor, g1r, be1r, g2r, be2r):
    n = x.shape[0]
    bn = 1000
    grid = (n // bn,)

    def body(x_r, xl_r, xr_r, ls0_r, ls1_r, num0_r, num1_r, den0_r, den1_r,
             wet_r, a_r, s_r, w1_r, b1_r, w2_r, b2_r, bo_r,
             g1_r, be1_r, g2_r, be2_r, out_r):
        xlv = xl_r[...]
        den16 = den0_r[...] + den1_r[...]
        deg = jnp.maximum(den16[:, 8:9], 1.0)
        la = (ls0_r[...] + ls1_r[...]) / deg
        lee = jnp.dot(la, wet_r[...], preferred_element_type=F32)
        z = xlv + xr_r[...] + lee
        a16 = jnp.exp(jnp.dot(_leaky(z), a_r[...], preferred_element_type=F32))
        den_e = jnp.dot(den16 + a16, s_r[...], preferred_element_type=F32)
        num_e = (num0_r[...] + num1_r[...]
                 + xlv * jnp.dot(a16, s_r[...], preferred_element_type=F32))
        attn = num_e / den_e + bo_r[...]
        h1 = _lnorm(x_r[...] + attn, g1_r[...], be1_r[...])
        p = jnp.dot(h1, w1_r[...], preferred_element_type=F32) + b1_r[...]
        f = jnp.dot(_gelu(p), w2_r[...], preferred_element_type=F32) + b2_r[...]
        out_r[...] = _lnorm(h1 + f, g2_r[...], be2_r[...])

    rows = pl.BlockSpec((bn, _HID), lambda i: (i, 0))
    rows16 = pl.BlockSpec((bn, 16), lambda i: (i, 0))
    full = pl.BlockSpec((_HID, _HID), lambda i: (0, 0))
    vec = pl.BlockSpec((1, _HID), lambda i: (0, 0))
    return pl.pallas_call(
        body,
        grid=grid,
        in_specs=[
            rows, rows, rows, rows, rows, rows, rows, rows16, rows16,
            full,
            pl.BlockSpec((_HID, 16), lambda i: (0, 0)),
            pl.BlockSpec((16, _HID), lambda i: (0, 0)),
            pl.BlockSpec((_HID, 4 * _HID), lambda i: (0, 0)),
            pl.BlockSpec((1, 4 * _HID), lambda i: (0, 0)),
            pl.BlockSpec((4 * _HID, _HID), lambda i: (0, 0)),
            vec, vec, vec, vec, vec, vec,
        ],
        out_specs=rows,
        out_shape=jax.ShapeDtypeStruct((n, _HID), F32),
    )(x, xl, xr, ls0, ls1, num0, num1, den0, den1,
      WeT, A, S, W1T, b1r, W2T, b2r, bor, g1r, be1r, g2r, be2r)


def kernel(x, edge_index, edge_attr, batch, Wl, bl, Wr, br, We, att,
           bias_out, W1, b1, W2, b2, g1, be1, g2, be2):
    n = x.shape[0]
    e = edge_index.shape[1]
    src = edge_index[0]
    dst = edge_index[1]

    np_ = -(-n // (_NS * _KB)) * (_NS * _KB)

    a8 = att.reshape(_HEADS, _HDIM)
    eye8 = jnp.eye(_HEADS, dtype=F32)
    amat = (a8[:, :, None] * eye8[:, None, :]).reshape(_HEADS * _HDIM, _HEADS)
    amat = jnp.pad(amat, ((0, 0), (0, 16 - _HEADS)))
    smat = jnp.pad(jnp.repeat(jnp.eye(_HEADS, dtype=F32), _HDIM, axis=1),
                   ((0, 16 - _HEADS), (0, 0)))

    xl, xr = _proj(x, Wl.T, bl.reshape(1, -1), Wr.T, br.reshape(1, -1))
    xlg, xrg, lsp = _sc_gather(e, np_)(xl, xr, src, dst, edge_attr)
    ae, msg = _edge_math(edge_attr, xlg, xrg, We.T, amat, smat)
    nump = _sc_scatter(e, np_)(dst, msg)
    denp = _sc_scatter_den(e, np_)(dst, ae.reshape(e * 16))[:, :, :16]
    return _finish(
        x, xl, xr, lsp[0, :n], lsp[1, :n], nump[0, :n], nump[1, :n],
        denp[0, :n], denp[1, :n],
        We.T, amat, smat, W1.T, b1.reshape(1, -1), W2.T, b2.reshape(1, -1),
        bias_out.reshape(1, -1), g1.reshape(1, -1), be1.reshape(1, -1),
        g2.reshape(1, -1), be2.reshape(1, -1))


# speedup vs baseline: 37.6258x; 1.1579x over previous
import functools

import jax
import jax.numpy as jnp
from jax import lax
from jax.experimental import pallas as pl
from jax.experimental.pallas import tpu as pltpu
from jax.experimental.pallas import tpu_sc as plsc

F32 = jnp.float32

_HID = 128
_HEADS = 8
_HDIM = 16

_NC = 2
_NS = 16
_NW = _NC * _NS

_KB = 80


def _leaky(z):
    return jnp.where(z >= 0, z, 0.2 * z)


def _lnorm(v, g, b):
    mu = jnp.mean(v, axis=-1, keepdims=True)
    var = jnp.mean((v - mu) * (v - mu), axis=-1, keepdims=True)
    return (v - mu) * lax.rsqrt(var + 1e-5) * g + b


def _gelu(v):
    return 0.5 * v * (1.0 + lax.erf(v * 0.7071067811865476))


def _proj(x, WlT, bl2, WrT, br2):
    n = x.shape[0]
    bn = 1000
    grid = (n // bn,)

    def body(x_r, wl_r, bl_r, wr_r, br_r, xl_r, xr_r):
        xv = x_r[...]
        xl_r[...] = jnp.dot(xv, wl_r[...], preferred_element_type=F32) + bl_r[...]
        xr_r[...] = jnp.dot(xv, wr_r[...], preferred_element_type=F32) + br_r[...]

    full = pl.BlockSpec((_HID, _HID), lambda i: (0, 0))
    vec = pl.BlockSpec((1, _HID), lambda i: (0, 0))
    rows = pl.BlockSpec((bn, _HID), lambda i: (i, 0))
    return pl.pallas_call(
        body,
        grid=grid,
        in_specs=[rows, full, vec, full, vec],
        out_specs=[rows, rows],
        out_shape=[
            jax.ShapeDtypeStruct((n, _HID), F32),
            jax.ShapeDtypeStruct((n, _HID), F32),
        ],
    )(x, WlT, bl2, WrT, br2)


def _sc_gather(e, np_):
    epw = e // _NW
    kbg = 40
    npair = epw // (2 * kbg)
    rpt = np_ // _NS
    mesh = plsc.VectorSubcoreMesh(core_axis_name="c", subcore_axis_name="s")

    @functools.partial(
        pl.kernel,
        out_type=(
            jax.ShapeDtypeStruct((e, _HID), F32),
            jax.ShapeDtypeStruct((e, _HID), F32),
            jax.ShapeDtypeStruct((_NC, np_, _HID), F32),
        ),
        mesh=mesh,
        scratch_types=(
            pltpu.VMEM((kbg,), jnp.int32),
            pltpu.VMEM((kbg,), jnp.int32),
            pltpu.VMEM((kbg,), jnp.int32),
            pltpu.VMEM((kbg,), jnp.int32),
            pltpu.VMEM((kbg, _HID), F32),
            pltpu.VMEM((kbg, _HID), F32),
            pltpu.VMEM((kbg, _HID), F32),
            pltpu.VMEM((kbg, _HID), F32),
            pltpu.VMEM((kbg, _HID), F32),
            pltpu.VMEM((kbg, _HID), F32),
            pltpu.VMEM_SHARED((np_, _HID), F32),
        ) + (pltpu.SemaphoreType.DMA,) * 10,
    )
    def k(xl_hbm, xr_hbm, src_hbm, dst_hbm, ea_hbm,
          xlg_out, xrg_out, ls_out,
          sv0, sv1, dv0, dv1, xlr0, xlr1, xrr0, xrr1, ear0, ear1, ls_sh,
          si0, si1, se0, se1, sg0, sg1, sw0, sw1, sa0, sa1):
        c = lax.axis_index("c")
        s = lax.axis_index("s")
        wid = c * _NS + s
        bufs = [
            (sv0, dv0, xlr0, xrr0, ear0, si0, se0, sg0, sw0, sa0),
            (sv1, dv1, xlr1, xrr1, ear1, si1, se1, sg1, sw1, sa1),
        ]

        def zrow(i, _):
            for h in range(_HID // 16):
                ear0[i, pl.ds(h * 16, 16)] = jnp.zeros((16,), F32)
            return 0
        lax.fori_loop(0, kbg, zrow, 0)

        def zchunk(j, _):
            pltpu.sync_copy(ear0, ls_sh.at[pl.ds(s * rpt + j * kbg, kbg)])
            return 0
        lax.fori_loop(0, rpt // kbg, zchunk, 0)
        plsc.subcore_barrier()

        base0 = wid * epw

        def load(g, b):
            sv, dv, _xl, _xr, ear, si, se, _sg, _sw, _sa = bufs[b]
            base = base0 + g * kbg
            pltpu.async_copy(src_hbm.at[pl.ds(base, kbg)], sv, si)
            pltpu.async_copy(dst_hbm.at[pl.ds(base, kbg)], dv, si)
            pltpu.async_copy(ea_hbm.at[pl.ds(base, kbg)], ear, se)

        def wait_idx(b):
            sv, dv, _xl, _xr, _e, si, _se, _sg, _sw, _sa = bufs[b]
            pltpu.make_async_copy(src_hbm.at[pl.ds(0, kbg)], sv, si).wait()
            pltpu.make_async_copy(dst_hbm.at[pl.ds(0, kbg)], dv, si).wait()

        def gathers(b):
            sv, dv, xlr, xrr, _e, _si, _se, sg, _sw, _sa = bufs[b]
            pltpu.async_copy(xl_hbm.at[sv], xlr, sg)
            pltpu.async_copy(xr_hbm.at[dv], xrr, sg)

        def drain_gathers(b):
            sv, dv, xlr, xrr, _e, _si, _se, sg, _sw, _sa = bufs[b]
            pltpu.make_async_copy(xl_hbm.at[sv], xlr, sg).wait()
            pltpu.make_async_copy(xr_hbm.at[dv], xrr, sg).wait()

        def outs(g, b):
            _sv, _dv, xlr, xrr, _e, _si, _se, _sg, sw, _sa = bufs[b]
            base = base0 + g * kbg
            pltpu.async_copy(xlr, xlg_out.at[pl.ds(base, kbg)], sw)
            pltpu.async_copy(xrr, xrg_out.at[pl.ds(base, kbg)], sw)

        def drain_outs(b):
            _sv, _dv, xlr, xrr, _e, _si, _se, _sg, sw, _sa = bufs[b]
            pltpu.make_async_copy(xlr, xlg_out.at[pl.ds(base0, kbg)], sw).wait()
            pltpu.make_async_copy(xrr, xrg_out.at[pl.ds(base0, kbg)], sw).wait()

        def drain_ea(b):
            _sv, _dv, _xl, _xr, ear, _si, se, _sg, _sw, _sa = bufs[b]
            pltpu.make_async_copy(ea_hbm.at[pl.ds(0, kbg)], ear, se).wait()

        def scat(b):
            _sv, dv, _xl, _xr, ear, _si, _se, _sg, _sw, sa = bufs[b]
            pltpu.async_copy(ear, ls_sh.at[dv], sa, add=True)

        def drain_scat(b):
            _sv, dv, _xl, _xr, ear, _si, _se, _sg, _sw, sa = bufs[b]
            pltpu.make_async_copy(ear, ls_sh.at[dv], sa).wait()

        load(0, 0)
        load(1, 1)

        def pair(i, _):
            g0 = 2 * i

            @pl.when(i > 0)
            def _():
                drain_outs(0)
            wait_idx(0)
            gathers(0)

            @pl.when(i > 0)
            def _():
                drain_outs(1)
            wait_idx(1)
            gathers(1)

            drain_gathers(0)
            outs(g0, 0)
            drain_ea(0)
            scat(0)

            drain_gathers(1)
            outs(g0 + 1, 1)
            drain_ea(1)
            scat(1)

            @pl.when(i < npair - 1)
            def _():
                drain_scat(0)
                load(g0 + 2, 0)
                drain_scat(1)
                load(g0 + 3, 1)
            return 0
        lax.fori_loop(0, npair, pair, 0)
        drain_outs(0)
        drain_outs(1)
        drain_scat(0)
        drain_scat(1)
        plsc.subcore_barrier()

        def wchunk(j, _):
            off = s * rpt + j * _KB
            pltpu.sync_copy(ls_sh.at[pl.ds(off, _KB)],
                            ls_out.at[c, pl.ds(off, _KB)])
            return 0
        lax.fori_loop(0, rpt // _KB, wchunk, 0)

    return k


def _edge_math(ea, xlg, xrg, WeT, A, S):
    e = ea.shape[0]
    be = 2000
    grid = (e // be,)

    def body(ea_r, xlg_r, xrg_r, wet_r, a_r, s_r, ae_r, msg_r):
        xlv = xlg_r[...]
        ee = jnp.dot(ea_r[...], wet_r[...], preferred_element_type=F32)
        z = xlv + xrg_r[...] + ee
        l16 = jnp.dot(_leaky(z), a_r[...], preferred_element_type=F32)
        aev = jnp.exp(l16)
        ae_r[...] = aev
        msg_r[...] = xlv * jnp.dot(aev, s_r[...], preferred_element_type=F32)

    rows = pl.BlockSpec((be, _HID), lambda i: (i, 0))
    return pl.pallas_call(
        body,
        grid=grid,
        in_specs=[
            rows, rows, rows,
            pl.BlockSpec((_HID, _HID), lambda i: (0, 0)),
            pl.BlockSpec((_HID, 16), lambda i: (0, 0)),
            pl.BlockSpec((16, _HID), lambda i: (0, 0)),
        ],
        out_specs=[pl.BlockSpec((be, 16), lambda i: (i, 0)), rows],
        out_shape=[
            jax.ShapeDtypeStruct((e, 16), F32),
            jax.ShapeDtypeStruct((e, _HID), F32),
        ],
    )(ea, xlg, xrg, WeT, A, S)


def _sc_scatter(e, np_):
    epw = e // _NW
    nblk = epw // _KB
    rpt = np_ // _NS
    mesh = plsc.VectorSubcoreMesh(core_axis_name="c", subcore_axis_name="s")

    @functools.partial(
        pl.kernel,
        out_type=jax.ShapeDtypeStruct((_NC, np_, _HID), F32),
        mesh=mesh,
        scratch_types=(
            pltpu.VMEM((_KB,), jnp.int32),
            pltpu.VMEM((_KB, _HID), F32),
            pltpu.VMEM_SHARED((np_, _HID), F32),
        ),
    )
    def k(dst_hbm, msg_hbm, num_out, dst_v, msgr, num_sh):
        c = lax.axis_index("c")
        s = lax.axis_index("s")
        wid = c * _NS + s

        def zrow(i, _):
            for h in range(_HID // 16):
                msgr[i, pl.ds(h * 16, 16)] = jnp.zeros((16,), F32)
            return 0
        lax.fori_loop(0, _KB, zrow, 0)

        def zchunk(j, _):
            off = s * rpt + j * _KB
            pltpu.sync_copy(msgr, num_sh.at[pl.ds(off, _KB)])
            return 0
        lax.fori_loop(0, rpt // _KB, zchunk, 0)
        plsc.subcore_barrier()

        base0 = wid * epw

        def body(i, _):
            base = base0 + i * _KB
            pltpu.sync_copy(dst_hbm.at[pl.ds(base, _KB)], dst_v)
            pltpu.sync_copy(msg_hbm.at[pl.ds(base, _KB)], msgr)
            pltpu.sync_copy(msgr, num_sh.at[dst_v], add=True)
            return 0
        lax.fori_loop(0, nblk, body, 0)
        plsc.subcore_barrier()

        def wchunk(j, _):
            off = s * rpt + j * _KB
            pltpu.sync_copy(num_sh.at[pl.ds(off, _KB)],
                            num_out.at[c, pl.ds(off, _KB)])
            return 0
        lax.fori_loop(0, rpt // _KB, wchunk, 0)

    return k


def _sc_scatter_den(e, np_):
    epw = e // _NW
    nblk = epw // _KB
    rpt = np_ // _NS
    mesh = plsc.VectorSubcoreMesh(core_axis_name="c", subcore_axis_name="s")

    @functools.partial(
        pl.kernel,
        out_type=jax.ShapeDtypeStruct((_NC, np_, _HID), F32),
        mesh=mesh,
        scratch_types=(
            pltpu.VMEM((_KB,), jnp.int32),
            pltpu.VMEM((_KB * 16,), F32),
            pltpu.VMEM((_KB, _HID), F32),
            pltpu.VMEM_SHARED((np_, _HID), F32),
        ),
    )
    def k(dst_hbm, aef_hbm, den_out, dst_v, aebuf, aer, den_sh):
        c = lax.axis_index("c")
        s = lax.axis_index("s")
        wid = c * _NS + s

        def zrow(i, _):
            for h in range(_HID // 16):
                aer[i, pl.ds(h * 16, 16)] = jnp.zeros((16,), F32)
            return 0
        lax.fori_loop(0, _KB, zrow, 0)

        def zchunk(j, _):
            off = s * rpt + j * _KB
            pltpu.sync_copy(aer, den_sh.at[pl.ds(off, _KB)])
            return 0
        lax.fori_loop(0, rpt // _KB, zchunk, 0)
        plsc.subcore_barrier()

        base0 = wid * epw

        def body(i, _):
            base = base0 + i * _KB
            pltpu.sync_copy(dst_hbm.at[pl.ds(base, _KB)], dst_v)
            pltpu.sync_copy(aef_hbm.at[pl.ds(base * 16, _KB * 16)], aebuf)
            for r in range(_KB):
                aer[r, pl.ds(0, 16)] = aebuf[pl.ds(r * 16, 16)]
            pltpu.sync_copy(aer, den_sh.at[dst_v], add=True)
            return 0
        lax.fori_loop(0, nblk, body, 0)
        plsc.subcore_barrier()

        def wchunk(j, _):
            off = s * rpt + j * _KB
            pltpu.sync_copy(den_sh.at[pl.ds(off, _KB)],
                            den_out.at[c, pl.ds(off, _KB)])
            return 0
        lax.fori_loop(0, rpt // _KB, wchunk, 0)

    return k


def _finish(x, xl, xr, ls0, ls1, num0, num1, den0, den1,
            WeT, A, S, W1T, b1r, W2T, b2r, bor, g1r, be1r, g2r, be2r):
    n = x.shape[0]
    bn = 1000
    grid = (n // bn,)

    def body(x_r, xl_r, xr_r, ls0_r, ls1_r, num0_r, num1_r, den0_r, den1_r,
             wet_r, a_r, s_r, w1_r, b1_r, w2_r, b2_r, bo_r,
             g1_r, be1_r, g2_r, be2_r, out_r):
        xlv = xl_r[...]
        den16 = den0_r[...] + den1_r[...]
        deg = jnp.maximum(den16[:, 8:9], 1.0)
        la = (ls0_r[...] + ls1_r[...]) / deg
        lee = jnp.dot(la, wet_r[...], preferred_element_type=F32)
        z = xlv + xr_r[...] + lee
        a16 = jnp.exp(jnp.dot(_leaky(z), a_r[...], preferred_element_type=F32))
        den_e = jnp.dot(den16 + a16, s_r[...], preferred_element_type=F32)
        num_e = (num0_r[...] + num1_r[...]
                 + xlv * jnp.dot(a16, s_r[...], preferred_element_type=F32))
        attn = num_e / den_e + bo_r[...]
        h1 = _lnorm(x_r[...] + attn, g1_r[...], be1_r[...])
        p = jnp.dot(h1, w1_r[...], preferred_element_type=F32) + b1_r[...]
        f = jnp.dot(_gelu(p), w2_r[...], preferred_element_type=F32) + b2_r[...]
        out_r[...] = _lnorm(h1 + f, g2_r[...], be2_r[...])

    rows = pl.BlockSpec((bn, _HID), lambda i: (i, 0))
    rows16 = pl.BlockSpec((bn, 16), lambda i: (i, 0))
    full = pl.BlockSpec((_HID, _HID), lambda i: (0, 0))
    vec = pl.BlockSpec((1, _HID), lambda i: (0, 0))
    return pl.pallas_call(
        body,
        grid=grid,
        in_specs=[
            rows, rows, rows, rows, rows, rows, rows, rows16, rows16,
            full,
            pl.BlockSpec((_HID, 16), lambda i: (0, 0)),
            pl.BlockSpec((16, _HID), lambda i: (0, 0)),
            pl.BlockSpec((_HID, 4 * _HID), lambda i: (0, 0)),
            pl.BlockSpec((1, 4 * _HID), lambda i: (0, 0)),
            pl.BlockSpec((4 * _HID, _HID), lambda i: (0, 0)),
            vec, vec, vec, vec, vec, vec,
        ],
        out_specs=rows,
        out_shape=jax.ShapeDtypeStruct((n, _HID), F32),
    )(x, xl, xr, ls0, ls1, num0, num1, den0, den1,
      WeT, A, S, W1T, b1r, W2T, b2r, bor, g1r, be1r, g2r, be2r)


def kernel(x, edge_index, edge_attr, batch, Wl, bl, Wr, br, We, att,
           bias_out, W1, b1, W2, b2, g1, be1, g2, be2):
    n = x.shape[0]
    e = edge_index.shape[1]
    src = edge_index[0]
    dst = edge_index[1]

    np_ = -(-n // (_NS * _KB)) * (_NS * _KB)

    a8 = att.reshape(_HEADS, _HDIM)
    eye8 = jnp.eye(_HEADS, dtype=F32)
    amat = (a8[:, :, None] * eye8[:, None, :]).reshape(_HEADS * _HDIM, _HEADS)
    amat = jnp.pad(amat, ((0, 0), (0, 16 - _HEADS)))
    smat = jnp.pad(jnp.repeat(jnp.eye(_HEADS, dtype=F32), _HDIM, axis=1),
                   ((0, 16 - _HEADS), (0, 0)))

    xl, xr = _proj(x, Wl.T, bl.reshape(1, -1), Wr.T, br.reshape(1, -1))
    xlg, xrg, lsp = _sc_gather(e, np_)(xl, xr, src, dst, edge_attr)
    ae, msg = _edge_math(edge_attr, xlg, xrg, We.T, amat, smat)
    nump = _sc_scatter(e, np_)(dst, msg)
    denp = _sc_scatter_den(e, np_)(dst, ae.reshape(e * 16))
    return _finish(
        x, xl, xr, lsp[0, :n], lsp[1, :n], nump[0, :n], nump[1, :n],
        denp[0, :n, :16], denp[1, :n, :16],
        We.T, amat, smat, W1.T, b1.reshape(1, -1), W2.T, b2.reshape(1, -1),
        bias_out.reshape(1, -1), g1.reshape(1, -1), be1.reshape(1, -1),
        g2.reshape(1, -1), be2.reshape(1, -1))


# speedup vs baseline: 42.5181x; 1.1300x over previous
import functools

import jax
import jax.numpy as jnp
from jax import lax
from jax.experimental import pallas as pl
from jax.experimental.pallas import tpu as pltpu
from jax.experimental.pallas import tpu_sc as plsc

F32 = jnp.float32

_HID = 128
_HEADS = 8
_HDIM = 16

_NC = 2
_NS = 16
_NW = _NC * _NS

_KB = 80


def _leaky(z):
    return jnp.where(z >= 0, z, 0.2 * z)


def _lnorm(v, g, b):
    mu = jnp.mean(v, axis=-1, keepdims=True)
    var = jnp.mean((v - mu) * (v - mu), axis=-1, keepdims=True)
    return (v - mu) * lax.rsqrt(var + 1e-5) * g + b


def _gelu(v):
    return 0.5 * v * (1.0 + lax.erf(v * 0.7071067811865476))


def _proj(x, WlT, bl2, WrT, br2):
    n = x.shape[0]
    bn = 1000
    grid = (n // bn,)

    def body(x_r, wl_r, bl_r, wr_r, br_r, xl_r, xr_r):
        xv = x_r[...]
        xl_r[...] = jnp.dot(xv, wl_r[...], preferred_element_type=F32) + bl_r[...]
        xr_r[...] = jnp.dot(xv, wr_r[...], preferred_element_type=F32) + br_r[...]

    full = pl.BlockSpec((_HID, _HID), lambda i: (0, 0))
    vec = pl.BlockSpec((1, _HID), lambda i: (0, 0))
    rows = pl.BlockSpec((bn, _HID), lambda i: (i, 0))
    return pl.pallas_call(
        body,
        grid=grid,
        in_specs=[rows, full, vec, full, vec],
        out_specs=[rows, rows],
        out_shape=[
            jax.ShapeDtypeStruct((n, _HID), F32),
            jax.ShapeDtypeStruct((n, _HID), F32),
        ],
    )(x, WlT, bl2, WrT, br2)


def _sc_gather(e, np_):
    epw = e // _NW
    kbg = 40
    npair = epw // (2 * kbg)
    rpt = np_ // _NS
    mesh = plsc.VectorSubcoreMesh(core_axis_name="c", subcore_axis_name="s")

    @functools.partial(
        pl.kernel,
        out_type=(
            jax.ShapeDtypeStruct((e, _HID), F32),
            jax.ShapeDtypeStruct((e, _HID), F32),
            jax.ShapeDtypeStruct((_NC, np_, _HID), F32),
        ),
        mesh=mesh,
        scratch_types=(
            pltpu.VMEM((kbg,), jnp.int32),
            pltpu.VMEM((kbg,), jnp.int32),
            pltpu.VMEM((kbg,), jnp.int32),
            pltpu.VMEM((kbg,), jnp.int32),
            pltpu.VMEM((kbg, _HID), F32),
            pltpu.VMEM((kbg, _HID), F32),
            pltpu.VMEM((kbg, _HID), F32),
            pltpu.VMEM((kbg, _HID), F32),
            pltpu.VMEM((kbg, _HID), F32),
            pltpu.VMEM((kbg, _HID), F32),
            pltpu.VMEM_SHARED((np_, _HID), F32),
        ) + (pltpu.SemaphoreType.DMA,) * 10,
    )
    def k(xl_hbm, xr_hbm, src_hbm, dst_hbm, ea_hbm,
          xlg_out, xrg_out, ls_out,
          sv0, sv1, dv0, dv1, xlr0, xlr1, xrr0, xrr1, ear0, ear1, ls_sh,
          si0, si1, se0, se1, sg0, sg1, sw0, sw1, sa0, sa1):
        c = lax.axis_index("c")
        s = lax.axis_index("s")
        wid = c * _NS + s
        bufs = [
            (sv0, dv0, xlr0, xrr0, ear0, si0, se0, sg0, sw0, sa0),
            (sv1, dv1, xlr1, xrr1, ear1, si1, se1, sg1, sw1, sa1),
        ]

        def zrow(i, _):
            for h in range(_HID // 16):
                ear0[i, pl.ds(h * 16, 16)] = jnp.zeros((16,), F32)
            return 0
        lax.fori_loop(0, kbg, zrow, 0)

        def zchunk(j, _):
            pltpu.sync_copy(ear0, ls_sh.at[pl.ds(s * rpt + j * kbg, kbg)])
            return 0
        lax.fori_loop(0, rpt // kbg, zchunk, 0)
        plsc.subcore_barrier()

        base0 = wid * epw

        def load(g, b):
            sv, dv, _xl, _xr, ear, si, se, _sg, _sw, _sa = bufs[b]
            base = base0 + g * kbg
            pltpu.async_copy(src_hbm.at[pl.ds(base, kbg)], sv, si)
            pltpu.async_copy(dst_hbm.at[pl.ds(base, kbg)], dv, si)
            pltpu.async_copy(ea_hbm.at[pl.ds(base, kbg)], ear, se)

        def wait_idx(b):
            sv, dv, _xl, _xr, _e, si, _se, _sg, _sw, _sa = bufs[b]
            pltpu.make_async_copy(src_hbm.at[pl.ds(0, kbg)], sv, si).wait()
            pltpu.make_async_copy(dst_hbm.at[pl.ds(0, kbg)], dv, si).wait()

        def gathers(b):
            sv, dv, xlr, xrr, _e, _si, _se, sg, _sw, _sa = bufs[b]
            pltpu.async_copy(xl_hbm.at[sv], xlr, sg)
            pltpu.async_copy(xr_hbm.at[dv], xrr, sg)

        def drain_gathers(b):
            sv, dv, xlr, xrr, _e, _si, _se, sg, _sw, _sa = bufs[b]
            pltpu.make_async_copy(xl_hbm.at[sv], xlr, sg).wait()
            pltpu.make_async_copy(xr_hbm.at[dv], xrr, sg).wait()

        def outs(g, b):
            _sv, _dv, xlr, xrr, _e, _si, _se, _sg, sw, _sa = bufs[b]
            base = base0 + g * kbg
            pltpu.async_copy(xlr, xlg_out.at[pl.ds(base, kbg)], sw)
            pltpu.async_copy(xrr, xrg_out.at[pl.ds(base, kbg)], sw)

        def drain_outs(b):
            _sv, _dv, xlr, xrr, _e, _si, _se, _sg, sw, _sa = bufs[b]
            pltpu.make_async_copy(xlr, xlg_out.at[pl.ds(base0, kbg)], sw).wait()
            pltpu.make_async_copy(xrr, xrg_out.at[pl.ds(base0, kbg)], sw).wait()

        def drain_ea(b):
            _sv, _dv, _xl, _xr, ear, _si, se, _sg, _sw, _sa = bufs[b]
            pltpu.make_async_copy(ea_hbm.at[pl.ds(0, kbg)], ear, se).wait()

        def scat(b):
            _sv, dv, _xl, _xr, ear, _si, _se, _sg, _sw, sa = bufs[b]
            pltpu.async_copy(ear, ls_sh.at[dv], sa, add=True)

        def drain_scat(b):
            _sv, dv, _xl, _xr, ear, _si, _se, _sg, _sw, sa = bufs[b]
            pltpu.make_async_copy(ear, ls_sh.at[dv], sa).wait()

        load(0, 0)
        load(1, 1)

        def pair(i, _):
            g0 = 2 * i

            @pl.when(i > 0)
            def _():
                drain_outs(0)
            wait_idx(0)
            gathers(0)

            @pl.when(i > 0)
            def _():
                drain_outs(1)
            wait_idx(1)
            gathers(1)

            drain_gathers(0)
            outs(g0, 0)
            drain_ea(0)
            scat(0)

            drain_gathers(1)
            outs(g0 + 1, 1)
            drain_ea(1)
            scat(1)

            @pl.when(i < npair - 1)
            def _():
                drain_scat(0)
                load(g0 + 2, 0)
                drain_scat(1)
                load(g0 + 3, 1)
            return 0
        lax.fori_loop(0, npair, pair, 0)
        drain_outs(0)
        drain_outs(1)
        drain_scat(0)
        drain_scat(1)
        plsc.subcore_barrier()

        def wchunk(j, _):
            off = s * rpt + j * _KB
            pltpu.sync_copy(ls_sh.at[pl.ds(off, _KB)],
                            ls_out.at[c, pl.ds(off, _KB)])
            return 0
        lax.fori_loop(0, rpt // _KB, wchunk, 0)

    return k


def _edge_math(ea, xlg, xrg, WeT, A, S):
    e = ea.shape[0]
    be = 2000
    grid = (e // be,)

    def body(ea_r, xlg_r, xrg_r, wet_r, a_r, s_r, ae_r, msg_r):
        xlv = xlg_r[...]
        ee = jnp.dot(ea_r[...], wet_r[...], preferred_element_type=F32)
        z = xlv + xrg_r[...] + ee
        l16 = jnp.dot(_leaky(z), a_r[...], preferred_element_type=F32)
        aev = jnp.exp(l16)
        ae_r[...] = aev
        msg_r[...] = xlv * jnp.dot(aev, s_r[...], preferred_element_type=F32)

    rows = pl.BlockSpec((be, _HID), lambda i: (i, 0))
    return pl.pallas_call(
        body,
        grid=grid,
        in_specs=[
            rows, rows, rows,
            pl.BlockSpec((_HID, _HID), lambda i: (0, 0)),
            pl.BlockSpec((_HID, 16), lambda i: (0, 0)),
            pl.BlockSpec((16, _HID), lambda i: (0, 0)),
        ],
        out_specs=[pl.BlockSpec((be, 16), lambda i: (i, 0)), rows],
        out_shape=[
            jax.ShapeDtypeStruct((e, 16), F32),
            jax.ShapeDtypeStruct((e, _HID), F32),
        ],
    )(ea, xlg, xrg, WeT, A, S)


def _sc_scatter(e, np_):
    epw = e // _NW
    kbs = 40
    npair = epw // (2 * kbs)
    rpt = np_ // _NS
    mesh = plsc.VectorSubcoreMesh(core_axis_name="c", subcore_axis_name="s")

    @functools.partial(
        pl.kernel,
        out_type=jax.ShapeDtypeStruct((_NC, np_, _HID), F32),
        mesh=mesh,
        scratch_types=(
            pltpu.VMEM((kbs,), jnp.int32),
            pltpu.VMEM((kbs,), jnp.int32),
            pltpu.VMEM((kbs, _HID), F32),
            pltpu.VMEM((kbs, _HID), F32),
            pltpu.VMEM_SHARED((np_, _HID), F32),
        ) + (pltpu.SemaphoreType.DMA,) * 4,
    )
    def k(dst_hbm, msg_hbm, num_out,
          dv0, dv1, mr0, mr1, num_sh, sl0, sl1, sa0, sa1):
        c = lax.axis_index("c")
        s = lax.axis_index("s")
        wid = c * _NS + s
        bufs = [(dv0, mr0, sl0, sa0), (dv1, mr1, sl1, sa1)]

        def zrow(i, _):
            for h in range(_HID // 16):
                mr0[i, pl.ds(h * 16, 16)] = jnp.zeros((16,), F32)
            return 0
        lax.fori_loop(0, kbs, zrow, 0)

        def zchunk(j, _):
            pltpu.sync_copy(mr0, num_sh.at[pl.ds(s * rpt + j * kbs, kbs)])
            return 0
        lax.fori_loop(0, rpt // kbs, zchunk, 0)
        plsc.subcore_barrier()

        base0 = wid * epw

        def load(g, b):
            dv, mr, sl, _sa = bufs[b]
            base = base0 + g * kbs
            pltpu.async_copy(dst_hbm.at[pl.ds(base, kbs)], dv, sl)
            pltpu.async_copy(msg_hbm.at[pl.ds(base, kbs)], mr, sl)

        def wait_load(b):
            dv, mr, sl, _sa = bufs[b]
            pltpu.make_async_copy(dst_hbm.at[pl.ds(0, kbs)], dv, sl).wait()
            pltpu.make_async_copy(msg_hbm.at[pl.ds(0, kbs)], mr, sl).wait()

        def scat(b):
            dv, mr, _sl, sa = bufs[b]
            pltpu.async_copy(mr, num_sh.at[dv], sa, add=True)

        def drain_scat(b):
            dv, mr, _sl, sa = bufs[b]
            pltpu.make_async_copy(mr, num_sh.at[dv], sa).wait()

        load(0, 0)
        load(1, 1)

        def pair(i, _):
            g0 = 2 * i
            wait_load(0)
            scat(0)
            wait_load(1)
            scat(1)

            @pl.when(i < npair - 1)
            def _():
                drain_scat(0)
                load(g0 + 2, 0)
                drain_scat(1)
                load(g0 + 3, 1)
            return 0
        lax.fori_loop(0, npair, pair, 0)
        drain_scat(0)
        drain_scat(1)
        plsc.subcore_barrier()

        def wchunk(j, _):
            off = s * rpt + j * _KB
            pltpu.sync_copy(num_sh.at[pl.ds(off, _KB)],
                            num_out.at[c, pl.ds(off, _KB)])
            return 0
        lax.fori_loop(0, rpt // _KB, wchunk, 0)

    return k


def _sc_scatter_den(e, np_):
    epw = e // _NW
    kbs = 40
    npair = epw // (2 * kbs)
    rpt = np_ // _NS
    mesh = plsc.VectorSubcoreMesh(core_axis_name="c", subcore_axis_name="s")

    @functools.partial(
        pl.kernel,
        out_type=jax.ShapeDtypeStruct((_NC, np_, _HID), F32),
        mesh=mesh,
        scratch_types=(
            pltpu.VMEM((kbs,), jnp.int32),
            pltpu.VMEM((kbs,), jnp.int32),
            pltpu.VMEM((kbs * 16,), F32),
            pltpu.VMEM((kbs * 16,), F32),
            pltpu.VMEM((kbs, _HID), F32),
            pltpu.VMEM((kbs, _HID), F32),
            pltpu.VMEM_SHARED((np_, _HID), F32),
        ) + (pltpu.SemaphoreType.DMA,) * 4,
    )
    def k(dst_hbm, aef_hbm, den_out,
          dv0, dv1, ab0, ab1, ar0, ar1, den_sh, sl0, sl1, sa0, sa1):
        c = lax.axis_index("c")
        s = lax.axis_index("s")
        wid = c * _NS + s
        bufs = [(dv0, ab0, ar0, sl0, sa0), (dv1, ab1, ar1, sl1, sa1)]

        def zrow(i, _):
            for h in range(_HID // 16):
                ar0[i, pl.ds(h * 16, 16)] = jnp.zeros((16,), F32)
                ar1[i, pl.ds(h * 16, 16)] = jnp.zeros((16,), F32)
            return 0
        lax.fori_loop(0, kbs, zrow, 0)

        def zchunk(j, _):
            pltpu.sync_copy(ar0, den_sh.at[pl.ds(s * rpt + j * kbs, kbs)])
            return 0
        lax.fori_loop(0, rpt // kbs, zchunk, 0)
        plsc.subcore_barrier()

        base0 = wid * epw

        def load(g, b):
            dv, ab, _ar, sl, _sa = bufs[b]
            base = base0 + g * kbs
            pltpu.async_copy(dst_hbm.at[pl.ds(base, kbs)], dv, sl)
            pltpu.async_copy(aef_hbm.at[pl.ds(base * 16, kbs * 16)], ab, sl)

        def wait_load(b):
            dv, ab, _ar, sl, _sa = bufs[b]
            pltpu.make_async_copy(dst_hbm.at[pl.ds(0, kbs)], dv, sl).wait()
            pltpu.make_async_copy(aef_hbm.at[pl.ds(0, kbs * 16)], ab, sl).wait()

        def unpack(b):
            _dv, ab, ar, _sl, _sa = bufs[b]
            for r in range(kbs):
                ar[r, pl.ds(0, 16)] = ab[pl.ds(r * 16, 16)]

        def scat(b):
            dv, _ab, ar, _sl, sa = bufs[b]
            pltpu.async_copy(ar, den_sh.at[dv], sa, add=True)

        def drain_scat(b):
            dv, _ab, ar, _sl, sa = bufs[b]
            pltpu.make_async_copy(ar, den_sh.at[dv], sa).wait()

        load(0, 0)
        load(1, 1)

        def pair(i, _):
            g0 = 2 * i
            wait_load(0)
            unpack(0)
            scat(0)
            wait_load(1)
            unpack(1)
            scat(1)

            @pl.when(i < npair - 1)
            def _():
                drain_scat(0)
                load(g0 + 2, 0)
                drain_scat(1)
                load(g0 + 3, 1)
            return 0
        lax.fori_loop(0, npair, pair, 0)
        drain_scat(0)
        drain_scat(1)
        plsc.subcore_barrier()

        def wchunk(j, _):
            off = s * rpt + j * _KB
            pltpu.sync_copy(den_sh.at[pl.ds(off, _KB)],
                            den_out.at[c, pl.ds(off, _KB)])
            return 0
        lax.fori_loop(0, rpt // _KB, wchunk, 0)

    return k


def _finish(x, xl, xr, ls0, ls1, num0, num1, den0, den1,
            WeT, A, S, W1T, b1r, W2T, b2r, bor, g1r, be1r, g2r, be2r):
    n = x.shape[0]
    bn = 1000
    grid = (n // bn,)

    def body(x_r, xl_r, xr_r, ls0_r, ls1_r, num0_r, num1_r, den0_r, den1_r,
             wet_r, a_r, s_r, w1_r, b1_r, w2_r, b2_r, bo_r,
             g1_r, be1_r, g2_r, be2_r, out_r):
        xlv = xl_r[...]
        den16 = den0_r[...] + den1_r[...]
        deg = jnp.maximum(den16[:, 8:9], 1.0)
        la = (ls0_r[...] + ls1_r[...]) / deg
        lee = jnp.dot(la, wet_r[...], preferred_element_type=F32)
        z = xlv + xr_r[...] + lee
        a16 = jnp.exp(jnp.dot(_leaky(z), a_r[...], preferred_element_type=F32))
        den_e = jnp.dot(den16 + a16, s_r[...], preferred_element_type=F32)
        num_e = (num0_r[...] + num1_r[...]
                 + xlv * jnp.dot(a16, s_r[...], preferred_element_type=F32))
        attn = num_e / den_e + bo_r[...]
        h1 = _lnorm(x_r[...] + attn, g1_r[...], be1_r[...])
        p = jnp.dot(h1, w1_r[...], preferred_element_type=F32) + b1_r[...]
        f = jnp.dot(_gelu(p), w2_r[...], preferred_element_type=F32) + b2_r[...]
        out_r[...] = _lnorm(h1 + f, g2_r[...], be2_r[...])

    rows = pl.BlockSpec((bn, _HID), lambda i: (i, 0))
    rows16 = pl.BlockSpec((bn, 16), lambda i: (i, 0))
    full = pl.BlockSpec((_HID, _HID), lambda i: (0, 0))
    vec = pl.BlockSpec((1, _HID), lambda i: (0, 0))
    return pl.pallas_call(
        body,
        grid=grid,
        in_specs=[
            rows, rows, rows, rows, rows, rows, rows, rows16, rows16,
            full,
            pl.BlockSpec((_HID, 16), lambda i: (0, 0)),
            pl.BlockSpec((16, _HID), lambda i: (0, 0)),
            pl.BlockSpec((_HID, 4 * _HID), lambda i: (0, 0)),
            pl.BlockSpec((1, 4 * _HID), lambda i: (0, 0)),
            pl.BlockSpec((4 * _HID, _HID), lambda i: (0, 0)),
            vec, vec, vec, vec, vec, vec,
        ],
        out_specs=rows,
        out_shape=jax.ShapeDtypeStruct((n, _HID), F32),
    )(x, xl, xr, ls0, ls1, num0, num1, den0, den1,
      WeT, A, S, W1T, b1r, W2T, b2r, bor, g1r, be1r, g2r, be2r)


def kernel(x, edge_index, edge_attr, batch, Wl, bl, Wr, br, We, att,
           bias_out, W1, b1, W2, b2, g1, be1, g2, be2):
    n = x.shape[0]
    e = edge_index.shape[1]
    src = edge_index[0]
    dst = edge_index[1]

    np_ = -(-n // (_NS * _KB)) * (_NS * _KB)

    a8 = att.reshape(_HEADS, _HDIM)
    eye8 = jnp.eye(_HEADS, dtype=F32)
    amat = (a8[:, :, None] * eye8[:, None, :]).reshape(_HEADS * _HDIM, _HEADS)
    amat = jnp.pad(amat, ((0, 0), (0, 16 - _HEADS)))
    smat = jnp.pad(jnp.repeat(jnp.eye(_HEADS, dtype=F32), _HDIM, axis=1),
                   ((0, 16 - _HEADS), (0, 0)))

    xl, xr = _proj(x, Wl.T, bl.reshape(1, -1), Wr.T, br.reshape(1, -1))
    xlg, xrg, lsp = _sc_gather(e, np_)(xl, xr, src, dst, edge_attr)
    ae, msg = _edge_math(edge_attr, xlg, xrg, We.T, amat, smat)
    nump = _sc_scatter(e, np_)(dst, msg)
    denp = _sc_scatter_den(e, np_)(dst, ae.reshape(e * 16))
    return _finish(
        x, xl, xr, lsp[0, :n], lsp[1, :n], nump[0, :n], nump[1, :n],
        denp[0, :n, :16], denp[1, :n, :16],
        We.T, amat, smat, W1.T, b1.reshape(1, -1), W2.T, b2.reshape(1, -1),
        bias_out.reshape(1, -1), g1.reshape(1, -1), be1.reshape(1, -1),
        g2.reshape(1, -1), be2.reshape(1, -1))


# speedup vs baseline: 47.6487x; 1.1207x over previous
import functools

import jax
import jax.numpy as jnp
from jax import lax
from jax.experimental import pallas as pl
from jax.experimental.pallas import tpu as pltpu
from jax.experimental.pallas import tpu_sc as plsc

F32 = jnp.float32

_HID = 128
_HEADS = 8
_HDIM = 16

_NC = 2
_NS = 16
_NW = _NC * _NS

_KB = 80


def _leaky(z):
    return jnp.where(z >= 0, z, 0.2 * z)


def _lnorm(v, g, b):
    mu = jnp.mean(v, axis=-1, keepdims=True)
    var = jnp.mean((v - mu) * (v - mu), axis=-1, keepdims=True)
    return (v - mu) * lax.rsqrt(var + 1e-5) * g + b


def _gelu(v):
    return 0.5 * v * (1.0 + lax.erf(v * 0.7071067811865476))


def _proj(x, WlT, bl2, WrT, br2):
    n = x.shape[0]
    bn = 1000
    grid = (n // bn,)

    def body(x_r, wl_r, bl_r, wr_r, br_r, xl_r, xr_r):
        xv = x_r[...]
        xl_r[...] = jnp.dot(xv, wl_r[...], preferred_element_type=F32) + bl_r[...]
        xr_r[...] = jnp.dot(xv, wr_r[...], preferred_element_type=F32) + br_r[...]

    full = pl.BlockSpec((_HID, _HID), lambda i: (0, 0))
    vec = pl.BlockSpec((1, _HID), lambda i: (0, 0))
    rows = pl.BlockSpec((bn, _HID), lambda i: (i, 0))
    return pl.pallas_call(
        body,
        grid=grid,
        in_specs=[rows, full, vec, full, vec],
        out_specs=[rows, rows],
        out_shape=[
            jax.ShapeDtypeStruct((n, _HID), F32),
            jax.ShapeDtypeStruct((n, _HID), F32),
        ],
    )(x, WlT, bl2, WrT, br2)


def _sc_gather(e, np_):
    epw = e // _NW
    kbg = 40
    npair = epw // (2 * kbg)
    rpt = np_ // _NS
    mesh = plsc.VectorSubcoreMesh(core_axis_name="c", subcore_axis_name="s")

    @functools.partial(
        pl.kernel,
        out_type=(
            jax.ShapeDtypeStruct((e, _HID), F32),
            jax.ShapeDtypeStruct((e, _HID), F32),
            jax.ShapeDtypeStruct((_NC, np_, _HID), F32),
        ),
        mesh=mesh,
        scratch_types=(
            pltpu.VMEM((kbg,), jnp.int32),
            pltpu.VMEM((kbg,), jnp.int32),
            pltpu.VMEM((kbg,), jnp.int32),
            pltpu.VMEM((kbg,), jnp.int32),
            pltpu.VMEM((kbg, _HID), F32),
            pltpu.VMEM((kbg, _HID), F32),
            pltpu.VMEM((kbg, _HID), F32),
            pltpu.VMEM((kbg, _HID), F32),
            pltpu.VMEM((kbg, _HID), F32),
            pltpu.VMEM((kbg, _HID), F32),
            pltpu.VMEM_SHARED((np_, _HID), F32),
        ) + (pltpu.SemaphoreType.DMA,) * 10,
    )
    def k(xl_hbm, xr_hbm, src_hbm, dst_hbm, ea_hbm,
          xlg_out, xrg_out, ls_out,
          sv0, sv1, dv0, dv1, xlr0, xlr1, xrr0, xrr1, ear0, ear1, ls_sh,
          si0, si1, se0, se1, sg0, sg1, sw0, sw1, sa0, sa1):
        c = lax.axis_index("c")
        s = lax.axis_index("s")
        wid = c * _NS + s
        bufs = [
            (sv0, dv0, xlr0, xrr0, ear0, si0, se0, sg0, sw0, sa0),
            (sv1, dv1, xlr1, xrr1, ear1, si1, se1, sg1, sw1, sa1),
        ]

        def zrow(i, _):
            for h in range(_HID // 16):
                ear0[i, pl.ds(h * 16, 16)] = jnp.zeros((16,), F32)
            return 0
        lax.fori_loop(0, kbg, zrow, 0)

        def zchunk(j, _):
            pltpu.sync_copy(ear0, ls_sh.at[pl.ds(s * rpt + j * kbg, kbg)])
            return 0
        lax.fori_loop(0, rpt // kbg, zchunk, 0)
        plsc.subcore_barrier()

        base0 = wid * epw

        def load(g, b):
            sv, dv, _xl, _xr, ear, si, se, _sg, _sw, _sa = bufs[b]
            base = base0 + g * kbg
            pltpu.async_copy(src_hbm.at[pl.ds(base, kbg)], sv, si)
            pltpu.async_copy(dst_hbm.at[pl.ds(base, kbg)], dv, si)
            pltpu.async_copy(ea_hbm.at[pl.ds(base, kbg)], ear, se)

        def wait_idx(b):
            sv, dv, _xl, _xr, _e, si, _se, _sg, _sw, _sa = bufs[b]
            pltpu.make_async_copy(src_hbm.at[pl.ds(0, kbg)], sv, si).wait()
            pltpu.make_async_copy(dst_hbm.at[pl.ds(0, kbg)], dv, si).wait()

        def gathers(b):
            sv, dv, xlr, xrr, _e, _si, _se, sg, _sw, _sa = bufs[b]
            pltpu.async_copy(xl_hbm.at[sv], xlr, sg)
            pltpu.async_copy(xr_hbm.at[dv], xrr, sg)

        def drain_gathers(b):
            sv, dv, xlr, xrr, _e, _si, _se, sg, _sw, _sa = bufs[b]
            pltpu.make_async_copy(xl_hbm.at[sv], xlr, sg).wait()
            pltpu.make_async_copy(xr_hbm.at[dv], xrr, sg).wait()

        def outs(g, b):
            _sv, _dv, xlr, xrr, _e, _si, _se, _sg, sw, _sa = bufs[b]
            base = base0 + g * kbg
            pltpu.async_copy(xlr, xlg_out.at[pl.ds(base, kbg)], sw)
            pltpu.async_copy(xrr, xrg_out.at[pl.ds(base, kbg)], sw)

        def drain_outs(b):
            _sv, _dv, xlr, xrr, _e, _si, _se, _sg, sw, _sa = bufs[b]
            pltpu.make_async_copy(xlr, xlg_out.at[pl.ds(base0, kbg)], sw).wait()
            pltpu.make_async_copy(xrr, xrg_out.at[pl.ds(base0, kbg)], sw).wait()

        def drain_ea(b):
            _sv, _dv, _xl, _xr, ear, _si, se, _sg, _sw, _sa = bufs[b]
            pltpu.make_async_copy(ea_hbm.at[pl.ds(0, kbg)], ear, se).wait()

        def scat(b):
            _sv, dv, _xl, _xr, ear, _si, _se, _sg, _sw, sa = bufs[b]
            pltpu.async_copy(ear, ls_sh.at[dv], sa, add=True)

        def drain_scat(b):
            _sv, dv, _xl, _xr, ear, _si, _se, _sg, _sw, sa = bufs[b]
            pltpu.make_async_copy(ear, ls_sh.at[dv], sa).wait()

        load(0, 0)
        load(1, 1)

        def pair(i, _):
            g0 = 2 * i

            @pl.when(i > 0)
            def _():
                drain_outs(0)
            wait_idx(0)
            gathers(0)

            @pl.when(i > 0)
            def _():
                drain_outs(1)
            wait_idx(1)
            gathers(1)

            drain_gathers(0)
            outs(g0, 0)
            drain_ea(0)
            scat(0)

            drain_gathers(1)
            outs(g0 + 1, 1)
            drain_ea(1)
            scat(1)

            @pl.when(i < npair - 1)
            def _():
                drain_scat(0)
                load(g0 + 2, 0)
                drain_scat(1)
                load(g0 + 3, 1)
            return 0
        lax.fori_loop(0, npair, pair, 0)
        drain_outs(0)
        drain_outs(1)
        drain_scat(0)
        drain_scat(1)
        plsc.subcore_barrier()

        def wchunk(j, _):
            off = s * rpt + j * _KB
            pltpu.sync_copy(ls_sh.at[pl.ds(off, _KB)],
                            ls_out.at[c, pl.ds(off, _KB)])
            return 0
        lax.fori_loop(0, rpt // _KB, wchunk, 0)

    return k


def _edge_math(ea, xlg, xrg, WeT, A, S):
    e = ea.shape[0]
    be = 2000
    grid = (e // be,)

    def body(ea_r, xlg_r, xrg_r, wet_r, a_r, s_r, ae_r, msg_r):
        xlv = xlg_r[...]
        ee = jnp.dot(ea_r[...], wet_r[...], preferred_element_type=F32)
        z = xlv + xrg_r[...] + ee
        l16 = jnp.dot(_leaky(z), a_r[...], preferred_element_type=F32)
        aev = jnp.exp(l16)
        ae_r[...] = aev
        msg_r[...] = xlv * jnp.dot(aev, s_r[...], preferred_element_type=F32)

    rows = pl.BlockSpec((be, _HID), lambda i: (i, 0))
    return pl.pallas_call(
        body,
        grid=grid,
        in_specs=[
            rows, rows, rows,
            pl.BlockSpec((_HID, _HID), lambda i: (0, 0)),
            pl.BlockSpec((_HID, 16), lambda i: (0, 0)),
            pl.BlockSpec((16, _HID), lambda i: (0, 0)),
        ],
        out_specs=[pl.BlockSpec((be, 16), lambda i: (i, 0)), rows],
        out_shape=[
            jax.ShapeDtypeStruct((e, 16), F32),
            jax.ShapeDtypeStruct((e, _HID), F32),
        ],
    )(ea, xlg, xrg, WeT, A, S)


def _sc_scatter(e, np_):
    epw = e // _NW
    kbs = 40
    nbuf = 5
    ngrp = epw // (nbuf * kbs)
    rpt = np_ // _NS
    mesh = plsc.VectorSubcoreMesh(core_axis_name="c", subcore_axis_name="s")

    @functools.partial(
        pl.kernel,
        out_type=jax.ShapeDtypeStruct((_NC, np_, _HID), F32),
        mesh=mesh,
        scratch_types=(pltpu.VMEM((kbs,), jnp.int32),) * nbuf
        + (pltpu.VMEM((kbs, _HID), F32),) * nbuf
        + (pltpu.VMEM_SHARED((np_, _HID), F32),)
        + (pltpu.SemaphoreType.DMA,) * (2 * nbuf),
    )
    def k(dst_hbm, msg_hbm, num_out, *scr):
        dvs = scr[0:nbuf]
        mrs = scr[nbuf:2 * nbuf]
        num_sh = scr[2 * nbuf]
        sls = scr[2 * nbuf + 1:3 * nbuf + 1]
        sas = scr[3 * nbuf + 1:4 * nbuf + 1]
        c = lax.axis_index("c")
        s = lax.axis_index("s")
        wid = c * _NS + s

        def zrow(i, _):
            for h in range(_HID // 16):
                mrs[0][i, pl.ds(h * 16, 16)] = jnp.zeros((16,), F32)
            return 0
        lax.fori_loop(0, kbs, zrow, 0)

        def zchunk(j, _):
            pltpu.sync_copy(mrs[0], num_sh.at[pl.ds(s * rpt + j * kbs, kbs)])
            return 0
        lax.fori_loop(0, rpt // kbs, zchunk, 0)
        plsc.subcore_barrier()

        base0 = wid * epw

        def load(g, b):
            base = base0 + g * kbs
            pltpu.async_copy(dst_hbm.at[pl.ds(base, kbs)], dvs[b], sls[b])
            pltpu.async_copy(msg_hbm.at[pl.ds(base, kbs)], mrs[b], sls[b])

        def wait_load(b):
            pltpu.make_async_copy(dst_hbm.at[pl.ds(0, kbs)], dvs[b], sls[b]).wait()
            pltpu.make_async_copy(msg_hbm.at[pl.ds(0, kbs)], mrs[b], sls[b]).wait()

        def scat(b):
            pltpu.async_copy(mrs[b], num_sh.at[dvs[b]], sas[b], add=True)

        def drain_scat(b):
            pltpu.make_async_copy(mrs[b], num_sh.at[dvs[b]], sas[b]).wait()

        for b in range(nbuf):
            load(b, b)

        def grp(i, _):
            g0 = nbuf * i
            for b in range(nbuf):
                wait_load(b)
                scat(b)

            @pl.when(i < ngrp - 1)
            def _():
                for b in range(nbuf):
                    drain_scat(b)
                    load(g0 + nbuf + b, b)
            return 0
        lax.fori_loop(0, ngrp, grp, 0)
        for b in range(nbuf):
            drain_scat(b)
        plsc.subcore_barrier()

        def wchunk(j, _):
            off = s * rpt + j * _KB
            pltpu.sync_copy(num_sh.at[pl.ds(off, _KB)],
                            num_out.at[c, pl.ds(off, _KB)])
            return 0
        lax.fori_loop(0, rpt // _KB, wchunk, 0)

    return k


def _sc_scatter_den(e, np_):
    epw = e // _NW
    kbs = 40
    nbuf = 5
    ngrp = epw // (nbuf * kbs)
    rpt = np_ // _NS
    mesh = plsc.VectorSubcoreMesh(core_axis_name="c", subcore_axis_name="s")

    @functools.partial(
        pl.kernel,
        out_type=jax.ShapeDtypeStruct((_NC, np_, _HID), F32),
        mesh=mesh,
        scratch_types=(pltpu.VMEM((kbs,), jnp.int32),) * nbuf
        + (pltpu.VMEM((kbs * 16,), F32),) * nbuf
        + (pltpu.VMEM((kbs, _HID), F32),) * nbuf
        + (pltpu.VMEM_SHARED((np_, _HID), F32),)
        + (pltpu.SemaphoreType.DMA,) * (2 * nbuf),
    )
    def k(dst_hbm, aef_hbm, den_out, *scr):
        dvs = scr[0:nbuf]
        abs_ = scr[nbuf:2 * nbuf]
        ars = scr[2 * nbuf:3 * nbuf]
        den_sh = scr[3 * nbuf]
        sls = scr[3 * nbuf + 1:4 * nbuf + 1]
        sas = scr[4 * nbuf + 1:5 * nbuf + 1]
        c = lax.axis_index("c")
        s = lax.axis_index("s")
        wid = c * _NS + s

        def zrow(i, _):
            for b in range(nbuf):
                for h in range(_HID // 16):
                    ars[b][i, pl.ds(h * 16, 16)] = jnp.zeros((16,), F32)
            return 0
        lax.fori_loop(0, kbs, zrow, 0)

        def zchunk(j, _):
            pltpu.sync_copy(ars[0], den_sh.at[pl.ds(s * rpt + j * kbs, kbs)])
            return 0
        lax.fori_loop(0, rpt // kbs, zchunk, 0)
        plsc.subcore_barrier()

        base0 = wid * epw

        def load(g, b):
            base = base0 + g * kbs
            pltpu.async_copy(dst_hbm.at[pl.ds(base, kbs)], dvs[b], sls[b])
            pltpu.async_copy(aef_hbm.at[pl.ds(base * 16, kbs * 16)], abs_[b],
                             sls[b])

        def wait_load(b):
            pltpu.make_async_copy(dst_hbm.at[pl.ds(0, kbs)], dvs[b], sls[b]).wait()
            pltpu.make_async_copy(aef_hbm.at[pl.ds(0, kbs * 16)], abs_[b],
                                  sls[b]).wait()

        def scat(b):
            pltpu.async_copy(ars[b], den_sh.at[dvs[b]], sas[b], add=True)

        def drain_scat(b):
            pltpu.make_async_copy(ars[b], den_sh.at[dvs[b]], sas[b]).wait()

        for b in range(nbuf):
            load(b, b)

        def grp(i, _):
            g0 = nbuf * i
            for b in range(nbuf):
                wait_load(b)
                for r in range(kbs):
                    ars[b][r, pl.ds(0, 16)] = abs_[b][pl.ds(r * 16, 16)]
                scat(b)

            @pl.when(i < ngrp - 1)
            def _():
                for b in range(nbuf):
                    drain_scat(b)
                    load(g0 + nbuf + b, b)
            return 0
        lax.fori_loop(0, ngrp, grp, 0)
        for b in range(nbuf):
            drain_scat(b)
        plsc.subcore_barrier()

        def wchunk(j, _):
            off = s * rpt + j * _KB
            pltpu.sync_copy(den_sh.at[pl.ds(off, _KB)],
                            den_out.at[c, pl.ds(off, _KB)])
            return 0
        lax.fori_loop(0, rpt // _KB, wchunk, 0)

    return k


def _finish(x, xl, xr, ls0, ls1, num0, num1, den0, den1,
            WeT, A, S, W1T, b1r, W2T, b2r, bor, g1r, be1r, g2r, be2r):
    n = x.shape[0]
    bn = 1000
    grid = (n // bn,)

    def body(x_r, xl_r, xr_r, ls0_r, ls1_r, num0_r, num1_r, den0_r, den1_r,
             wet_r, a_r, s_r, w1_r, b1_r, w2_r, b2_r, bo_r,
             g1_r, be1_r, g2_r, be2_r, out_r):
        xlv = xl_r[...]
        den16 = den0_r[...] + den1_r[...]
        deg = jnp.maximum(den16[:, 8:9], 1.0)
        la = (ls0_r[...] + ls1_r[...]) / deg
        lee = jnp.dot(la, wet_r[...], preferred_element_type=F32)
        z = xlv + xr_r[...] + lee
        a16 = jnp.exp(jnp.dot(_leaky(z), a_r[...], preferred_element_type=F32))
        den_e = jnp.dot(den16 + a16, s_r[...], preferred_element_type=F32)
        num_e = (num0_r[...] + num1_r[...]
                 + xlv * jnp.dot(a16, s_r[...], preferred_element_type=F32))
        attn = num_e / den_e + bo_r[...]
        h1 = _lnorm(x_r[...] + attn, g1_r[...], be1_r[...])
        p = jnp.dot(h1, w1_r[...], preferred_element_type=F32) + b1_r[...]
        f = jnp.dot(_gelu(p), w2_r[...], preferred_element_type=F32) + b2_r[...]
        out_r[...] = _lnorm(h1 + f, g2_r[...], be2_r[...])

    rows = pl.BlockSpec((bn, _HID), lambda i: (i, 0))
    rows16 = pl.BlockSpec((bn, 16), lambda i: (i, 0))
    full = pl.BlockSpec((_HID, _HID), lambda i: (0, 0))
    vec = pl.BlockSpec((1, _HID), lambda i: (0, 0))
    return pl.pallas_call(
        body,
        grid=grid,
        in_specs=[
            rows, rows, rows, rows, rows, rows, rows, rows16, rows16,
            full,
            pl.BlockSpec((_HID, 16), lambda i: (0, 0)),
            pl.BlockSpec((16, _HID), lambda i: (0, 0)),
            pl.BlockSpec((_HID, 4 * _HID), lambda i: (0, 0)),
            pl.BlockSpec((1, 4 * _HID), lambda i: (0, 0)),
            pl.BlockSpec((4 * _HID, _HID), lambda i: (0, 0)),
            vec, vec, vec, vec, vec, vec,
        ],
        out_specs=rows,
        out_shape=jax.ShapeDtypeStruct((n, _HID), F32),
    )(x, xl, xr, ls0, ls1, num0, num1, den0, den1,
      WeT, A, S, W1T, b1r, W2T, b2r, bor, g1r, be1r, g2r, be2r)


def kernel(x, edge_index, edge_attr, batch, Wl, bl, Wr, br, We, att,
           bias_out, W1, b1, W2, b2, g1, be1, g2, be2):
    n = x.shape[0]
    e = edge_index.shape[1]
    src = edge_index[0]
    dst = edge_index[1]

    np_ = -(-n // (_NS * _KB)) * (_NS * _KB)

    a8 = att.reshape(_HEADS, _HDIM)
    eye8 = jnp.eye(_HEADS, dtype=F32)
    amat = (a8[:, :, None] * eye8[:, None, :]).reshape(_HEADS * _HDIM, _HEADS)
    amat = jnp.pad(amat, ((0, 0), (0, 16 - _HEADS)))
    smat = jnp.pad(jnp.repeat(jnp.eye(_HEADS, dtype=F32), _HDIM, axis=1),
                   ((0, 16 - _HEADS), (0, 0)))

    xl, xr = _proj(x, Wl.T, bl.reshape(1, -1), Wr.T, br.reshape(1, -1))
    xlg, xrg, lsp = _sc_gather(e, np_)(xl, xr, src, dst, edge_attr)
    ae, msg = _edge_math(edge_attr, xlg, xrg, We.T, amat, smat)
    nump = _sc_scatter(e, np_)(dst, msg)
    denp = _sc_scatter_den(e, np_)(dst, ae.reshape(e * 16))
    return _finish(
        x, xl, xr, lsp[0, :n], lsp[1, :n], nump[0, :n], nump[1, :n],
        denp[0, :n, :16], denp[1, :n, :16],
        We.T, amat, smat, W1.T, b1.reshape(1, -1), W2.T, b2.reshape(1, -1),
        bias_out.reshape(1, -1), g1.reshape(1, -1), be1.reshape(1, -1),
        g2.reshape(1, -1), be2.reshape(1, -1))


# speedup vs baseline: 49.0217x; 1.0288x over previous
import functools

import jax
import jax.numpy as jnp
from jax import lax
from jax.experimental import pallas as pl
from jax.experimental.pallas import tpu as pltpu
from jax.experimental.pallas import tpu_sc as plsc

F32 = jnp.float32

_HID = 128
_HEADS = 8
_HDIM = 16

_NC = 2
_NS = 16
_NW = _NC * _NS

_KB = 80


def _leaky(z):
    return jnp.where(z >= 0, z, 0.2 * z)


def _lnorm(v, g, b):
    mu = jnp.mean(v, axis=-1, keepdims=True)
    var = jnp.mean((v - mu) * (v - mu), axis=-1, keepdims=True)
    return (v - mu) * lax.rsqrt(var + 1e-5) * g + b


def _gelu(v):
    return 0.5 * v * (1.0 + lax.erf(v * 0.7071067811865476))


def _proj(x, WlT, bl2, WrT, br2):
    n = x.shape[0]
    bn = 1000
    grid = (n // bn,)

    def body(x_r, wl_r, bl_r, wr_r, br_r, xl_r, xr_r):
        xv = x_r[...]
        xl_r[...] = jnp.dot(xv, wl_r[...], preferred_element_type=F32) + bl_r[...]
        xr_r[...] = jnp.dot(xv, wr_r[...], preferred_element_type=F32) + br_r[...]

    full = pl.BlockSpec((_HID, _HID), lambda i: (0, 0))
    vec = pl.BlockSpec((1, _HID), lambda i: (0, 0))
    rows = pl.BlockSpec((bn, _HID), lambda i: (i, 0))
    return pl.pallas_call(
        body,
        grid=grid,
        in_specs=[rows, full, vec, full, vec],
        out_specs=[rows, rows],
        out_shape=[
            jax.ShapeDtypeStruct((n, _HID), F32),
            jax.ShapeDtypeStruct((n, _HID), F32),
        ],
    )(x, WlT, bl2, WrT, br2)


def _sc_gather(e, np_):
    epw = e // _NW
    kbg = 40
    npair = epw // (2 * kbg)
    rpt = np_ // _NS
    mesh = plsc.VectorSubcoreMesh(core_axis_name="c", subcore_axis_name="s")

    @functools.partial(
        pl.kernel,
        out_type=(
            jax.ShapeDtypeStruct((e, _HID), F32),
            jax.ShapeDtypeStruct((e, _HID), F32),
            jax.ShapeDtypeStruct((_NC, np_, _HID), F32),
        ),
        mesh=mesh,
        scratch_types=(
            pltpu.VMEM((kbg,), jnp.int32),
            pltpu.VMEM((kbg,), jnp.int32),
            pltpu.VMEM((kbg,), jnp.int32),
            pltpu.VMEM((kbg,), jnp.int32),
            pltpu.VMEM((kbg, _HID), F32),
            pltpu.VMEM((kbg, _HID), F32),
            pltpu.VMEM((kbg, _HID), F32),
            pltpu.VMEM((kbg, _HID), F32),
            pltpu.VMEM((kbg, _HID), F32),
            pltpu.VMEM((kbg, _HID), F32),
            pltpu.VMEM_SHARED((np_, _HID), F32),
        ) + (pltpu.SemaphoreType.DMA,) * 10,
    )
    def k(xl_hbm, xr_hbm, src_hbm, dst_hbm, ea_hbm,
          xlg_out, xrg_out, ls_out,
          sv0, sv1, dv0, dv1, xlr0, xlr1, xrr0, xrr1, ear0, ear1, ls_sh,
          si0, si1, se0, se1, sg0, sg1, sw0, sw1, sa0, sa1):
        c = lax.axis_index("c")
        s = lax.axis_index("s")
        wid = c * _NS + s
        bufs = [
            (sv0, dv0, xlr0, xrr0, ear0, si0, se0, sg0, sw0, sa0),
            (sv1, dv1, xlr1, xrr1, ear1, si1, se1, sg1, sw1, sa1),
        ]

        def zrow(i, _):
            for h in range(_HID // 16):
                ear0[i, pl.ds(h * 16, 16)] = jnp.zeros((16,), F32)
            return 0
        lax.fori_loop(0, kbg, zrow, 0)

        def zchunk(j, _):
            pltpu.sync_copy(ear0, ls_sh.at[pl.ds(s * rpt + j * kbg, kbg)])
            return 0
        lax.fori_loop(0, rpt // kbg, zchunk, 0)
        plsc.subcore_barrier()

        base0 = wid * epw

        def load(g, b):
            sv, dv, _xl, _xr, ear, si, se, _sg, _sw, _sa = bufs[b]
            base = base0 + g * kbg
            pltpu.async_copy(src_hbm.at[pl.ds(base, kbg)], sv, si)
            pltpu.async_copy(dst_hbm.at[pl.ds(base, kbg)], dv, si)
            pltpu.async_copy(ea_hbm.at[pl.ds(base, kbg)], ear, se)

        def wait_idx(b):
            sv, dv, _xl, _xr, _e, si, _se, _sg, _sw, _sa = bufs[b]
            pltpu.make_async_copy(src_hbm.at[pl.ds(0, kbg)], sv, si).wait()
            pltpu.make_async_copy(dst_hbm.at[pl.ds(0, kbg)], dv, si).wait()

        def gathers(b):
            sv, dv, xlr, xrr, _e, _si, _se, sg, _sw, _sa = bufs[b]
            pltpu.async_copy(xl_hbm.at[sv], xlr, sg)
            pltpu.async_copy(xr_hbm.at[dv], xrr, sg)

        def drain_gathers(b):
            sv, dv, xlr, xrr, _e, _si, _se, sg, _sw, _sa = bufs[b]
            pltpu.make_async_copy(xl_hbm.at[sv], xlr, sg).wait()
            pltpu.make_async_copy(xr_hbm.at[dv], xrr, sg).wait()

        def outs(g, b):
            _sv, _dv, xlr, xrr, _e, _si, _se, _sg, sw, _sa = bufs[b]
            base = base0 + g * kbg
            pltpu.async_copy(xlr, xlg_out.at[pl.ds(base, kbg)], sw)
            pltpu.async_copy(xrr, xrg_out.at[pl.ds(base, kbg)], sw)

        def drain_outs(b):
            _sv, _dv, xlr, xrr, _e, _si, _se, _sg, sw, _sa = bufs[b]
            pltpu.make_async_copy(xlr, xlg_out.at[pl.ds(base0, kbg)], sw).wait()
            pltpu.make_async_copy(xrr, xrg_out.at[pl.ds(base0, kbg)], sw).wait()

        def drain_ea(b):
            _sv, _dv, _xl, _xr, ear, _si, se, _sg, _sw, _sa = bufs[b]
            pltpu.make_async_copy(ea_hbm.at[pl.ds(0, kbg)], ear, se).wait()

        def scat(b):
            _sv, dv, _xl, _xr, ear, _si, _se, _sg, _sw, sa = bufs[b]
            pltpu.async_copy(ear, ls_sh.at[dv], sa, add=True)

        def drain_scat(b):
            _sv, dv, _xl, _xr, ear, _si, _se, _sg, _sw, sa = bufs[b]
            pltpu.make_async_copy(ear, ls_sh.at[dv], sa).wait()

        load(0, 0)
        load(1, 1)

        def pair(i, _):
            g0 = 2 * i

            @pl.when(i > 0)
            def _():
                drain_outs(0)
            wait_idx(0)
            gathers(0)

            @pl.when(i > 0)
            def _():
                drain_outs(1)
            wait_idx(1)
            gathers(1)

            drain_gathers(0)
            outs(g0, 0)
            drain_ea(0)
            scat(0)

            drain_gathers(1)
            outs(g0 + 1, 1)
            drain_ea(1)
            scat(1)

            @pl.when(i < npair - 1)
            def _():
                drain_scat(0)
                load(g0 + 2, 0)
                drain_scat(1)
                load(g0 + 3, 1)
            return 0
        lax.fori_loop(0, npair, pair, 0)
        drain_outs(0)
        drain_outs(1)
        drain_scat(0)
        drain_scat(1)
        plsc.subcore_barrier()

        def wchunk(j, _):
            off = s * rpt + j * _KB
            pltpu.sync_copy(ls_sh.at[pl.ds(off, _KB)],
                            ls_out.at[c, pl.ds(off, _KB)])
            return 0
        lax.fori_loop(0, rpt // _KB, wchunk, 0)

    return k


def _edge_math(ea, xlg, xrg, WeT, A, S):
    e = ea.shape[0]
    be = 4000
    grid = (e // be,)

    def body(ea_r, xlg_r, xrg_r, wet_r, a_r, s_r, ae_r, msg_r):
        xlv = xlg_r[...]
        ee = jnp.dot(ea_r[...], wet_r[...], preferred_element_type=F32)
        z = xlv + xrg_r[...] + ee
        l16 = jnp.dot(_leaky(z), a_r[...], preferred_element_type=F32)
        aev = jnp.exp(l16)
        ae_r[...] = aev
        msg_r[...] = xlv * jnp.dot(aev, s_r[...], preferred_element_type=F32)

    rows = pl.BlockSpec((be, _HID), lambda i: (i, 0))
    return pl.pallas_call(
        body,
        grid=grid,
        in_specs=[
            rows, rows, rows,
            pl.BlockSpec((_HID, _HID), lambda i: (0, 0)),
            pl.BlockSpec((_HID, 16), lambda i: (0, 0)),
            pl.BlockSpec((16, _HID), lambda i: (0, 0)),
        ],
        out_specs=[pl.BlockSpec((be, 16), lambda i: (i, 0)), rows],
        out_shape=[
            jax.ShapeDtypeStruct((e, 16), F32),
            jax.ShapeDtypeStruct((e, _HID), F32),
        ],
    )(ea, xlg, xrg, WeT, A, S)


def _sc_scatter(e, np_):
    epw = e // _NW
    kbs = 40
    nbuf = 5
    ngrp = epw // (nbuf * kbs)
    rpt = np_ // _NS
    mesh = plsc.VectorSubcoreMesh(core_axis_name="c", subcore_axis_name="s")

    @functools.partial(
        pl.kernel,
        out_type=jax.ShapeDtypeStruct((_NC, np_, _HID), F32),
        mesh=mesh,
        scratch_types=(pltpu.VMEM((kbs,), jnp.int32),) * nbuf
        + (pltpu.VMEM((kbs, _HID), F32),) * nbuf
        + (pltpu.VMEM_SHARED((np_, _HID), F32),)
        + (pltpu.SemaphoreType.DMA,) * (2 * nbuf),
    )
    def k(dst_hbm, msg_hbm, num_out, *scr):
        dvs = scr[0:nbuf]
        mrs = scr[nbuf:2 * nbuf]
        num_sh = scr[2 * nbuf]
        sls = scr[2 * nbuf + 1:3 * nbuf + 1]
        sas = scr[3 * nbuf + 1:4 * nbuf + 1]
        c = lax.axis_index("c")
        s = lax.axis_index("s")
        wid = c * _NS + s

        def zrow(i, _):
            for h in range(_HID // 16):
                mrs[0][i, pl.ds(h * 16, 16)] = jnp.zeros((16,), F32)
            return 0
        lax.fori_loop(0, kbs, zrow, 0)

        def zchunk(j, _):
            pltpu.sync_copy(mrs[0], num_sh.at[pl.ds(s * rpt + j * kbs, kbs)])
            return 0
        lax.fori_loop(0, rpt // kbs, zchunk, 0)
        plsc.subcore_barrier()

        base0 = wid * epw

        def load(g, b):
            base = base0 + g * kbs
            pltpu.async_copy(dst_hbm.at[pl.ds(base, kbs)], dvs[b], sls[b])
            pltpu.async_copy(msg_hbm.at[pl.ds(base, kbs)], mrs[b], sls[b])

        def wait_load(b):
            pltpu.make_async_copy(dst_hbm.at[pl.ds(0, kbs)], dvs[b], sls[b]).wait()
            pltpu.make_async_copy(msg_hbm.at[pl.ds(0, kbs)], mrs[b], sls[b]).wait()

        def scat(b):
            pltpu.async_copy(mrs[b], num_sh.at[dvs[b]], sas[b], add=True)

        def drain_scat(b):
            pltpu.make_async_copy(mrs[b], num_sh.at[dvs[b]], sas[b]).wait()

        for b in range(nbuf):
            load(b, b)

        def grp(i, _):
            g0 = nbuf * i
            for b in range(nbuf):
                wait_load(b)
                scat(b)

            @pl.when(i < ngrp - 1)
            def _():
                for b in range(nbuf):
                    drain_scat(b)
                    load(g0 + nbuf + b, b)
            return 0
        lax.fori_loop(0, ngrp, grp, 0)
        for b in range(nbuf):
            drain_scat(b)
        plsc.subcore_barrier()

        def wchunk(j, _):
            off = s * rpt + j * _KB
            pltpu.sync_copy(num_sh.at[pl.ds(off, _KB)],
                            num_out.at[c, pl.ds(off, _KB)])
            return 0
        lax.fori_loop(0, rpt // _KB, wchunk, 0)

    return k


def _sc_scatter_den(e, np_):
    epw = e // _NW
    kbs = 40
    nbuf = 5
    ngrp = epw // (nbuf * kbs)
    rpt = np_ // _NS
    mesh = plsc.VectorSubcoreMesh(core_axis_name="c", subcore_axis_name="s")

    @functools.partial(
        pl.kernel,
        out_type=jax.ShapeDtypeStruct((_NC, np_, _HID), F32),
        mesh=mesh,
        scratch_types=(pltpu.VMEM((kbs,), jnp.int32),) * nbuf
        + (pltpu.VMEM((kbs * 16,), F32),) * nbuf
        + (pltpu.VMEM((kbs, _HID), F32),) * nbuf
        + (pltpu.VMEM_SHARED((np_, _HID), F32),)
        + (pltpu.SemaphoreType.DMA,) * (2 * nbuf),
    )
    def k(dst_hbm, aef_hbm, den_out, *scr):
        dvs = scr[0:nbuf]
        abs_ = scr[nbuf:2 * nbuf]
        ars = scr[2 * nbuf:3 * nbuf]
        den_sh = scr[3 * nbuf]
        sls = scr[3 * nbuf + 1:4 * nbuf + 1]
        sas = scr[4 * nbuf + 1:5 * nbuf + 1]
        c = lax.axis_index("c")
        s = lax.axis_index("s")
        wid = c * _NS + s

        def zrow(i, _):
            for b in range(nbuf):
                for h in range(_HID // 16):
                    ars[b][i, pl.ds(h * 16, 16)] = jnp.zeros((16,), F32)
            return 0
        lax.fori_loop(0, kbs, zrow, 0)

        def zchunk(j, _):
            pltpu.sync_copy(ars[0], den_sh.at[pl.ds(s * rpt + j * kbs, kbs)])
            return 0
        lax.fori_loop(0, rpt // kbs, zchunk, 0)
        plsc.subcore_barrier()

        base0 = wid * epw

        def load(g, b):
            base = base0 + g * kbs
            pltpu.async_copy(dst_hbm.at[pl.ds(base, kbs)], dvs[b], sls[b])
            pltpu.async_copy(aef_hbm.at[pl.ds(base * 16, kbs * 16)], abs_[b],
                             sls[b])

        def wait_load(b):
            pltpu.make_async_copy(dst_hbm.at[pl.ds(0, kbs)], dvs[b], sls[b]).wait()
            pltpu.make_async_copy(aef_hbm.at[pl.ds(0, kbs * 16)], abs_[b],
                                  sls[b]).wait()

        def scat(b):
            pltpu.async_copy(ars[b], den_sh.at[dvs[b]], sas[b], add=True)

        def drain_scat(b):
            pltpu.make_async_copy(ars[b], den_sh.at[dvs[b]], sas[b]).wait()

        for b in range(nbuf):
            load(b, b)

        def grp(i, _):
            g0 = nbuf * i
            for b in range(nbuf):
                wait_load(b)
                for r in range(kbs):
                    ars[b][r, pl.ds(0, 16)] = abs_[b][pl.ds(r * 16, 16)]
                scat(b)

            @pl.when(i < ngrp - 1)
            def _():
                for b in range(nbuf):
                    drain_scat(b)
                    load(g0 + nbuf + b, b)
            return 0
        lax.fori_loop(0, ngrp, grp, 0)
        for b in range(nbuf):
            drain_scat(b)
        plsc.subcore_barrier()

        def wchunk(j, _):
            off = s * rpt + j * _KB
            pltpu.sync_copy(den_sh.at[pl.ds(off, _KB)],
                            den_out.at[c, pl.ds(off, _KB)])
            return 0
        lax.fori_loop(0, rpt // _KB, wchunk, 0)

    return k


def _finish(x, xl, xr, ls0, ls1, num0, num1, den0, den1,
            WeT, A, S, W1T, b1r, W2T, b2r, bor, g1r, be1r, g2r, be2r):
    n = x.shape[0]
    bn = 1000
    grid = (n // bn,)

    def body(x_r, xl_r, xr_r, ls0_r, ls1_r, num0_r, num1_r, den0_r, den1_r,
             wet_r, a_r, s_r, w1_r, b1_r, w2_r, b2_r, bo_r,
             g1_r, be1_r, g2_r, be2_r, out_r):
        xlv = xl_r[...]
        den16 = den0_r[...] + den1_r[...]
        deg = jnp.maximum(den16[:, 8:9], 1.0)
        la = (ls0_r[...] + ls1_r[...]) / deg
        lee = jnp.dot(la, wet_r[...], preferred_element_type=F32)
        z = xlv + xr_r[...] + lee
        a16 = jnp.exp(jnp.dot(_leaky(z), a_r[...], preferred_element_type=F32))
        den_e = jnp.dot(den16 + a16, s_r[...], preferred_element_type=F32)
        num_e = (num0_r[...] + num1_r[...]
                 + xlv * jnp.dot(a16, s_r[...], preferred_element_type=F32))
        attn = num_e / den_e + bo_r[...]
        h1 = _lnorm(x_r[...] + attn, g1_r[...], be1_r[...])
        p = jnp.dot(h1, w1_r[...], preferred_element_type=F32) + b1_r[...]
        f = jnp.dot(_gelu(p), w2_r[...], preferred_element_type=F32) + b2_r[...]
        out_r[...] = _lnorm(h1 + f, g2_r[...], be2_r[...])

    rows = pl.BlockSpec((bn, _HID), lambda i: (i, 0))
    rows16 = pl.BlockSpec((bn, 16), lambda i: (i, 0))
    full = pl.BlockSpec((_HID, _HID), lambda i: (0, 0))
    vec = pl.BlockSpec((1, _HID), lambda i: (0, 0))
    return pl.pallas_call(
        body,
        grid=grid,
        in_specs=[
            rows, rows, rows, rows, rows, rows, rows, rows16, rows16,
            full,
            pl.BlockSpec((_HID, 16), lambda i: (0, 0)),
            pl.BlockSpec((16, _HID), lambda i: (0, 0)),
            pl.BlockSpec((_HID, 4 * _HID), lambda i: (0, 0)),
            pl.BlockSpec((1, 4 * _HID), lambda i: (0, 0)),
            pl.BlockSpec((4 * _HID, _HID), lambda i: (0, 0)),
            vec, vec, vec, vec, vec, vec,
        ],
        out_specs=rows,
        out_shape=jax.ShapeDtypeStruct((n, _HID), F32),
    )(x, xl, xr, ls0, ls1, num0, num1, den0, den1,
      WeT, A, S, W1T, b1r, W2T, b2r, bor, g1r, be1r, g2r, be2r)


def kernel(x, edge_index, edge_attr, batch, Wl, bl, Wr, br, We, att,
           bias_out, W1, b1, W2, b2, g1, be1, g2, be2):
    n = x.shape[0]
    e = edge_index.shape[1]
    src = edge_index[0]
    dst = edge_index[1]

    np_ = -(-n // (_NS * _KB)) * (_NS * _KB)

    a8 = att.reshape(_HEADS, _HDIM)
    eye8 = jnp.eye(_HEADS, dtype=F32)
    amat = (a8[:, :, None] * eye8[:, None, :]).reshape(_HEADS * _HDIM, _HEADS)
    amat = jnp.pad(amat, ((0, 0), (0, 16 - _HEADS)))
    smat = jnp.pad(jnp.repeat(jnp.eye(_HEADS, dtype=F32), _HDIM, axis=1),
                   ((0, 16 - _HEADS), (0, 0)))

    xl, xr = _proj(x, Wl.T, bl.reshape(1, -1), Wr.T, br.reshape(1, -1))
    xlg, xrg, lsp = _sc_gather(e, np_)(xl, xr, src, dst, edge_attr)
    ae, msg = _edge_math(edge_attr, xlg, xrg, We.T, amat, smat)
    nump = _sc_scatter(e, np_)(dst, msg)
    denp = _sc_scatter_den(e, np_)(dst, ae.reshape(e * 16))
    return _finish(
        x, xl, xr, lsp[0, :n], lsp[1, :n], nump[0, :n], nump[1, :n],
        denp[0, :n, :16], denp[1, :n, :16],
        We.T, amat, smat, W1.T, b1.reshape(1, -1), W2.T, b2.reshape(1, -1),
        bias_out.reshape(1, -1), g1.reshape(1, -1), be1.reshape(1, -1),
        g2.reshape(1, -1), be2.reshape(1, -1))
